# Initial kernel scaffold; baseline (speedup 1.0000x reference)
import functools

import jax
import jax.numpy as jnp
from jax import lax
from jax.experimental import pallas as pl
from jax.experimental.pallas import tpu as pltpu
from jax.experimental.pallas import tpu_sc as plsc

N_NODES = 10000
D_IN = 256
D_HID = 256
D_LAT = 128

N_PAD = 10016
E_PAD = 163840
NB = 80
BB = 128
ROWS_PER_SUB = N_PAD // 16

_mesh = plsc.VectorSubcoreMesh(core_axis_name="c", subcore_axis_name="s")


@functools.partial(
    pl.kernel,
    out_type=jax.ShapeDtypeStruct((32, N_PAD), jnp.float32),
    mesh=_mesh,
    scratch_types=[
        pltpu.VMEM((NB, BB), jnp.int32),
        pltpu.VMEM((N_PAD,), jnp.float32),
    ],
)
def _hist_kernel(dst3, out, dst_v, hist):
    c = lax.axis_index("c")
    s = lax.axis_index("s")
    wid = c * 16 + s
    pltpu.sync_copy(dst3.at[s], dst_v)

    @pl.loop(0, N_PAD, step=16)
    def _zero(i):
        hist[pl.ds(i, 16)] = jnp.zeros((16,), jnp.float32)

    ones = jnp.ones((16,), jnp.float32)

    @pl.loop(0, NB // 2)
    def _row(j):
        @pl.loop(0, BB, step=16)
        def _vec(k):
            idx = dst_v[j + c * (NB // 2), pl.ds(k, 16)]
            plsc.addupdate_scatter(hist, [idx], ones)

    pltpu.sync_copy(hist, out.at[wid])


@functools.partial(
    pl.kernel,
    out_type=[
        jax.ShapeDtypeStruct((N_PAD, 128), jnp.float32),
        jax.ShapeDtypeStruct((N_PAD, 128), jnp.float32),
    ],
    mesh=_mesh,
    scratch_types=[
        pltpu.VMEM((NB, BB), jnp.int32),
        pltpu.VMEM((NB, BB), jnp.int32),
        pltpu.VMEM((BB, 128), jnp.float32),
        pltpu.VMEM((BB, 128), jnp.float32),
        pltpu.VMEM_SHARED((N_PAD, 128), jnp.float32),
        pltpu.SemaphoreType.DMA,
        pltpu.SemaphoreType.DMA,
    ],
)
def _conv_kernel(hsA, hsB, src3, dst3, outA, outB, src_v, dst_v, rows0, rows1,
                 acc, gsem, ssem):
    c = lax.axis_index("c")
    s = lax.axis_index("s")
    base = s * ROWS_PER_SUB

    @pl.loop(0, BB)
    def _zr(i):
        @pl.loop(0, 128, step=16)
        def _zc(j):
            rows0[i, pl.ds(j, 16)] = jnp.zeros((16,), jnp.float32)

    @pl.loop(0, ROWS_PER_SUB // BB)
    def _za(k):
        pltpu.sync_copy(rows0, acc.at[pl.ds(base + k * BB, BB)])

    pltpu.sync_copy(
        rows0.at[pl.ds(0, ROWS_PER_SUB % BB)],
        acc.at[pl.ds(base + (ROWS_PER_SUB // BB) * BB, ROWS_PER_SUB % BB)],
    )

    pltpu.sync_copy(src3.at[s], src_v)
    pltpu.sync_copy(dst3.at[s], dst_v)
    plsc.subcore_barrier()

    def work(hs_ref):
        pltpu.async_copy(hs_ref.at[src_v.at[0]], rows0, gsem).wait()

        @pl.loop(0, NB // 2 - 1)
        def _pair(jh):
            j = jh * 2
            g1 = pltpu.async_copy(hs_ref.at[src_v.at[j + 1]], rows1, gsem)
            pltpu.async_copy(rows0, acc.at[dst_v.at[j]], ssem, add=True).wait()
            g1.wait()
            g0 = pltpu.async_copy(hs_ref.at[src_v.at[j + 2]], rows0, gsem)
            pltpu.async_copy(rows1, acc.at[dst_v.at[j + 1]], ssem, add=True).wait()
            g0.wait()

        g1 = pltpu.async_copy(hs_ref.at[src_v.at[NB - 1]], rows1, gsem)
        pltpu.async_copy(rows0, acc.at[dst_v.at[NB - 2]], ssem, add=True).wait()
        g1.wait()
        pltpu.async_copy(rows1, acc.at[dst_v.at[NB - 1]], ssem, add=True).wait()

    @pl.when(c == 0)
    def _w0():
        work(hsA)

    @pl.when(c == 1)
    def _w1():
        work(hsB)

    plsc.subcore_barrier()

    @pl.when(c == 0)
    def _o0():
        pltpu.sync_copy(acc.at[pl.ds(base, ROWS_PER_SUB)],
                        outA.at[pl.ds(base, ROWS_PER_SUB)])

    @pl.when(c == 1)
    def _o1():
        pltpu.sync_copy(acc.at[pl.ds(base, ROWS_PER_SUB)],
                        outB.at[pl.ds(base, ROWS_PER_SUB)])


_BM = 1000
_HIGH = jax.lax.Precision.HIGHEST


def _deg_body(p_ref, dis_ref):
    deg = jnp.sum(p_ref[...], axis=0, keepdims=True) + 1.0
    dis_ref[...] = lax.rsqrt(deg)


def _degree_kernel(partials):
    return pl.pallas_call(
        _deg_body,
        out_shape=jax.ShapeDtypeStruct((1, N_PAD), jnp.float32),
    )(partials)


def _m1_body(x_ref, w_ref, dis_ref, a_ref, b_ref):
    hs = jax.lax.dot_general(x_ref[...], w_ref[...], (((1,), (0,)), ((), ())),
                             precision=_HIGH,
                             preferred_element_type=jnp.float32)
    hs = hs * dis_ref[...]
    a_ref[...] = hs[:, :128]
    b_ref[...] = hs[:, 128:]


def _m1_kernel(x, W1, dis_col):
    return pl.pallas_call(
        _m1_body,
        grid=(N_NODES // _BM,),
        in_specs=[
            pl.BlockSpec((_BM, D_IN), lambda i: (i, 0)),
            pl.BlockSpec((D_IN, D_HID), lambda i: (0, 0)),
            pl.BlockSpec((_BM, 1), lambda i: (i, 0)),
        ],
        out_specs=[
            pl.BlockSpec((_BM, 128), lambda i: (i, 0)),
            pl.BlockSpec((_BM, 128), lambda i: (i, 0)),
        ],
        out_shape=[
            jax.ShapeDtypeStruct((N_NODES, 128), jnp.float32),
            jax.ShapeDtypeStruct((N_NODES, 128), jnp.float32),
        ],
    )(x, W1, dis_col)


def _e1_body(aggA_ref, aggB_ref, hsA_ref, hsB_ref, dis_ref, b1_ref, w_ref,
             a_ref, b_ref):
    dis = dis_ref[...]
    hidden = jnp.concatenate(
        [(aggA_ref[...] + hsA_ref[...]) * dis,
         (aggB_ref[...] + hsB_ref[...]) * dis], axis=1)
    hidden = jnp.maximum(hidden + b1_ref[...], 0.0)
    hs2 = jax.lax.dot_general(hidden, w_ref[...], (((1,), (0,)), ((), ())),
                              precision=_HIGH,
                              preferred_element_type=jnp.float32)
    hs2 = hs2 * dis
    a_ref[...] = hs2[:, :128]
    b_ref[...] = hs2[:, 128:]


def _e1_kernel(aggA, aggB, hsA, hsB, dis_col, b1_row, Wcat):
    blk = lambda i: (i, 0)
    return pl.pallas_call(
        _e1_body,
        grid=(N_NODES // _BM,),
        in_specs=[
            pl.BlockSpec((_BM, 128), blk),
            pl.BlockSpec((_BM, 128), blk),
            pl.BlockSpec((_BM, 128), blk),
            pl.BlockSpec((_BM, 128), blk),
            pl.BlockSpec((_BM, 1), blk),
            pl.BlockSpec((1, D_HID), lambda i: (0, 0)),
            pl.BlockSpec((D_HID, 2 * D_LAT), lambda i: (0, 0)),
        ],
        out_specs=[
            pl.BlockSpec((_BM, 128), blk),
            pl.BlockSpec((_BM, 128), blk),
        ],
        out_shape=[
            jax.ShapeDtypeStruct((N_NODES, 128), jnp.float32),
            jax.ShapeDtypeStruct((N_NODES, 128), jnp.float32),
        ],
    )(aggA, aggB, hsA, hsB, dis_col, b1_row, Wcat)


def _e2_body(aggA_ref, aggB_ref, hsA_ref, hsB_ref, dis_ref, bmu_ref, bls_ref,
             eps_ref, mu_ref, ls_ref, z_ref):
    dis = dis_ref[...]
    mu = (aggA_ref[...] + hsA_ref[...]) * dis + bmu_ref[...]
    ls = (aggB_ref[...] + hsB_ref[...]) * dis + bls_ref[...]
    mu_ref[...] = mu
    ls_ref[...] = ls
    z_ref[...] = mu + eps_ref[...] * jnp.exp(ls)


def _e2_kernel(aggA, aggB, hsA, hsB, dis_col, bmu_row, bls_row, eps):
    blk = lambda i: (i, 0)
    return pl.pallas_call(
        _e2_body,
        grid=(N_NODES // _BM,),
        in_specs=[
            pl.BlockSpec((_BM, 128), blk),
            pl.BlockSpec((_BM, 128), blk),
            pl.BlockSpec((_BM, 128), blk),
            pl.BlockSpec((_BM, 128), blk),
            pl.BlockSpec((_BM, 1), blk),
            pl.BlockSpec((1, D_LAT), lambda i: (0, 0)),
            pl.BlockSpec((1, D_LAT), lambda i: (0, 0)),
            pl.BlockSpec((_BM, D_LAT), blk),
        ],
        out_specs=[pl.BlockSpec((_BM, D_LAT), blk)] * 3,
        out_shape=[jax.ShapeDtypeStruct((N_NODES, D_LAT), jnp.float32)] * 3,
    )(aggA, aggB, hsA, hsB, dis_col, bmu_row, bls_row, eps)


def _dec_body(zi_ref, zj_ref, o_ref):
    o_ref[...] = jax.lax.dot_general(
        zi_ref[...], zj_ref[...], (((1,), (1,)), ((), ())),
        precision=_HIGH, preferred_element_type=jnp.float32)


def _decoder_kernel(z):
    return pl.pallas_call(
        _dec_body,
        grid=(N_NODES // _BM, N_NODES // _BM),
        in_specs=[
            pl.BlockSpec((_BM, D_LAT), lambda i, j: (i, 0)),
            pl.BlockSpec((_BM, D_LAT), lambda i, j: (j, 0)),
        ],
        out_specs=pl.BlockSpec((_BM, _BM), lambda i, j: (i, j)),
        out_shape=jax.ShapeDtypeStruct((N_NODES, N_NODES), jnp.float32),
        compiler_params=pltpu.CompilerParams(
            dimension_semantics=("parallel", "parallel")),
    )(z)


def kernel(x, edge_index, W1, b1, W_mu, b_mu, W_ls, b_ls, eps):
    src = edge_index[0].astype(jnp.int32)
    dst = edge_index[1].astype(jnp.int32)

    n_extra = E_PAD - src.shape[0]
    ar = jnp.arange(n_extra, dtype=jnp.int32)
    pad_src = (ar * 131) % N_NODES
    pad_dst = N_NODES + (ar % 16)
    src3 = jnp.concatenate([src, pad_src]).reshape(16, NB, BB)
    dst3 = jnp.concatenate([dst, pad_dst]).reshape(16, NB, BB)

    partials = _hist_kernel(dst3)
    dis_row = _degree_kernel(partials)
    dis_col = dis_row.reshape(N_PAD, 1)[:N_NODES]

    hsA1, hsB1 = _m1_kernel(x, W1, dis_col)
    aggA1, aggB1 = _conv_kernel(hsA1, hsB1, src3, dst3)
    Wcat = jnp.concatenate([W_mu, W_ls], axis=1)
    hsA2, hsB2 = _e1_kernel(aggA1[:N_NODES], aggB1[:N_NODES], hsA1, hsB1,
                            dis_col, b1.reshape(1, -1), Wcat)
    aggA2, aggB2 = _conv_kernel(hsA2, hsB2, src3, dst3)
    mu, logstd, z = _e2_kernel(aggA2[:N_NODES], aggB2[:N_NODES], hsA2, hsB2,
                               dis_col, b_mu.reshape(1, -1),
                               b_ls.reshape(1, -1), eps)
    adj = _decoder_kernel(z)
    return (adj, mu, logstd)


# speedup vs baseline: 8.2280x; 8.2280x over previous
import dataclasses
import functools

import jax
import jax.numpy as jnp
from jax import lax
from jax.experimental import pallas as pl
from jax.experimental.pallas import tpu as pltpu
from jax.experimental.pallas import tpu_sc as plsc

N_NODES = 10000
D_IN = 256
D_HID = 256
D_LAT = 128

N_PAD = 10112
E_PAD = 163840
NB = 80
BB = 128
ROWS_PER_SUB = N_PAD // 16


def _hist_body(dst3, out, dst_v, hist):
    c = lax.axis_index("c")
    s = lax.axis_index("s")
    wid = c * 16 + s
    pltpu.sync_copy(dst3.at[s], dst_v)

    @pl.loop(0, N_PAD, step=16)
    def _zero(i):
        hist[pl.ds(i, 16)] = jnp.zeros((16,), jnp.float32)

    ones = jnp.ones((16,), jnp.float32)

    @pl.loop(0, NB // 2)
    def _row(j):
        @pl.loop(0, BB, step=16)
        def _vec(k):
            idx = dst_v[j + c * (NB // 2), pl.ds(k, 16)]
            plsc.addupdate_scatter(hist, [idx], ones)

    pltpu.sync_copy(hist, out.at[wid])


def _conv_body(hsA, hsB, src3, dst3, outA, outB, src_v, dst_v, rows0, rows1,
               acc, gsem, ssem):
    c = lax.axis_index("c")
    s = lax.axis_index("s")
    base = s * ROWS_PER_SUB

    @pl.loop(0, BB)
    def _zr(i):
        @pl.loop(0, 128, step=16)
        def _zc(j):
            rows0[i, pl.ds(j, 16)] = jnp.zeros((16,), jnp.float32)

    @pl.loop(0, ROWS_PER_SUB // BB)
    def _za(k):
        pltpu.sync_copy(rows0, acc.at[pl.ds(base + k * BB, BB)])

    pltpu.sync_copy(
        rows0.at[pl.ds(0, ROWS_PER_SUB % BB)],
        acc.at[pl.ds(base + (ROWS_PER_SUB // BB) * BB, ROWS_PER_SUB % BB)],
    )

    pltpu.sync_copy(src3.at[s], src_v)
    pltpu.sync_copy(dst3.at[s], dst_v)
    plsc.subcore_barrier()

    def work(hs_ref):
        @pl.loop(0, NB)
        def _batch(j):
            pltpu.sync_copy(hs_ref.at[src_v.at[j]], rows0)
            pltpu.sync_copy(rows0, acc.at[dst_v.at[j]], add=True)

    @pl.when(c == 0)
    def _w0():
        work(hsA)

    @pl.when(c == 1)
    def _w1():
        work(hsB)

    plsc.subcore_barrier()

    @pl.when(c == 0)
    def _o0():
        pltpu.sync_copy(acc.at[pl.ds(base, ROWS_PER_SUB)],
                        outA.at[pl.ds(base, ROWS_PER_SUB)])

    @pl.when(c == 1)
    def _o1():
        pltpu.sync_copy(acc.at[pl.ds(base, ROWS_PER_SUB)],
                        outB.at[pl.ds(base, ROWS_PER_SUB)])


@functools.cache
def _sc_kernels():
    mesh = plsc.VectorSubcoreMesh(core_axis_name="c", subcore_axis_name="s")
    cp = pltpu.CompilerParams()
    if "needs_layout_passes" in pltpu.CompilerParams.__dataclass_fields__:
        cp = dataclasses.replace(cp, needs_layout_passes=False)
    hist = functools.partial(
        pl.kernel,
        out_type=jax.ShapeDtypeStruct((32, N_PAD), jnp.float32),
        mesh=mesh,
        compiler_params=cp,
        scratch_types=[
            pltpu.VMEM((NB, BB), jnp.int32),
            pltpu.VMEM((N_PAD,), jnp.float32),
        ],
    )(_hist_body)
    conv = functools.partial(
        pl.kernel,
        out_type=[
            jax.ShapeDtypeStruct((N_PAD, 128), jnp.float32),
            jax.ShapeDtypeStruct((N_PAD, 128), jnp.float32),
        ],
        mesh=mesh,
        scratch_types=[
            pltpu.VMEM((NB, BB), jnp.int32),
            pltpu.VMEM((NB, BB), jnp.int32),
            pltpu.VMEM((BB, 128), jnp.float32),
            pltpu.VMEM((BB, 128), jnp.float32),
            pltpu.VMEM_SHARED((N_PAD, 128), jnp.float32),
            pltpu.SemaphoreType.DMA,
            pltpu.SemaphoreType.DMA,
        ],
    )(_conv_body)
    return hist, conv


_BM = 1000
_HIGH = jax.lax.Precision.HIGHEST


def _deg_body(p_ref, dis_ref):
    deg = jnp.sum(p_ref[...], axis=0, keepdims=True) + 1.0
    dis_ref[...] = lax.rsqrt(deg)


def _degree_kernel(partials):
    return pl.pallas_call(
        _deg_body,
        out_shape=jax.ShapeDtypeStruct((1, N_PAD), jnp.float32),
    )(partials)


def _m1_body(x_ref, w_ref, dis_ref, a_ref, b_ref):
    hs = jax.lax.dot_general(x_ref[...], w_ref[...], (((1,), (0,)), ((), ())),
                             precision=_HIGH,
                             preferred_element_type=jnp.float32)
    hs = hs * dis_ref[...]
    a_ref[...] = hs[:, :128]
    b_ref[...] = hs[:, 128:]


def _m1_kernel(x, W1, dis_col):
    return pl.pallas_call(
        _m1_body,
        grid=(N_NODES // _BM,),
        in_specs=[
            pl.BlockSpec((_BM, D_IN), lambda i: (i, 0)),
            pl.BlockSpec((D_IN, D_HID), lambda i: (0, 0)),
            pl.BlockSpec((_BM, 1), lambda i: (i, 0)),
        ],
        out_specs=[
            pl.BlockSpec((_BM, 128), lambda i: (i, 0)),
            pl.BlockSpec((_BM, 128), lambda i: (i, 0)),
        ],
        out_shape=[
            jax.ShapeDtypeStruct((N_NODES, 128), jnp.float32),
            jax.ShapeDtypeStruct((N_NODES, 128), jnp.float32),
        ],
    )(x, W1, dis_col)


def _e1_body(aggA_ref, aggB_ref, hsA_ref, hsB_ref, dis_ref, b1_ref, w_ref,
             a_ref, b_ref):
    dis = dis_ref[...]
    hidden = jnp.concatenate(
        [(aggA_ref[...] + hsA_ref[...]) * dis,
         (aggB_ref[...] + hsB_ref[...]) * dis], axis=1)
    hidden = jnp.maximum(hidden + b1_ref[...], 0.0)
    hs2 = jax.lax.dot_general(hidden, w_ref[...], (((1,), (0,)), ((), ())),
                              precision=_HIGH,
                              preferred_element_type=jnp.float32)
    hs2 = hs2 * dis
    a_ref[...] = hs2[:, :128]
    b_ref[...] = hs2[:, 128:]


def _e1_kernel(aggA, aggB, hsA, hsB, dis_col, b1_row, Wcat):
    blk = lambda i: (i, 0)
    return pl.pallas_call(
        _e1_body,
        grid=(N_NODES // _BM,),
        in_specs=[
            pl.BlockSpec((_BM, 128), blk),
            pl.BlockSpec((_BM, 128), blk),
            pl.BlockSpec((_BM, 128), blk),
            pl.BlockSpec((_BM, 128), blk),
            pl.BlockSpec((_BM, 1), blk),
            pl.BlockSpec((1, D_HID), lambda i: (0, 0)),
            pl.BlockSpec((D_HID, 2 * D_LAT), lambda i: (0, 0)),
        ],
        out_specs=[
            pl.BlockSpec((_BM, 128), blk),
            pl.BlockSpec((_BM, 128), blk),
        ],
        out_shape=[
            jax.ShapeDtypeStruct((N_NODES, 128), jnp.float32),
            jax.ShapeDtypeStruct((N_NODES, 128), jnp.float32),
        ],
    )(aggA, aggB, hsA, hsB, dis_col, b1_row, Wcat)


def _e2_body(aggA_ref, aggB_ref, hsA_ref, hsB_ref, dis_ref, bmu_ref, bls_ref,
             eps_ref, mu_ref, ls_ref, z_ref):
    dis = dis_ref[...]
    mu = (aggA_ref[...] + hsA_ref[...]) * dis + bmu_ref[...]
    ls = (aggB_ref[...] + hsB_ref[...]) * dis + bls_ref[...]
    mu_ref[...] = mu
    ls_ref[...] = ls
    z_ref[...] = mu + eps_ref[...] * jnp.exp(ls)


def _e2_kernel(aggA, aggB, hsA, hsB, dis_col, bmu_row, bls_row, eps):
    blk = lambda i: (i, 0)
    return pl.pallas_call(
        _e2_body,
        grid=(N_NODES // _BM,),
        in_specs=[
            pl.BlockSpec((_BM, 128), blk),
            pl.BlockSpec((_BM, 128), blk),
            pl.BlockSpec((_BM, 128), blk),
            pl.BlockSpec((_BM, 128), blk),
            pl.BlockSpec((_BM, 1), blk),
            pl.BlockSpec((1, D_LAT), lambda i: (0, 0)),
            pl.BlockSpec((1, D_LAT), lambda i: (0, 0)),
            pl.BlockSpec((_BM, D_LAT), blk),
        ],
        out_specs=[pl.BlockSpec((_BM, D_LAT), blk)] * 3,
        out_shape=[jax.ShapeDtypeStruct((N_NODES, D_LAT), jnp.float32)] * 3,
    )(aggA, aggB, hsA, hsB, dis_col, bmu_row, bls_row, eps)


def _dec_body(zi_ref, zj_ref, o_ref):
    o_ref[...] = jax.lax.dot_general(
        zi_ref[...], zj_ref[...], (((1,), (1,)), ((), ())),
        precision=_HIGH, preferred_element_type=jnp.float32)


_BD = 1024


def _decoder_kernel(z):
    nblk = (N_NODES + _BD - 1) // _BD
    return pl.pallas_call(
        _dec_body,
        grid=(nblk, nblk),
        in_specs=[
            pl.BlockSpec((_BD, D_LAT), lambda i, j: (i, 0)),
            pl.BlockSpec((_BD, D_LAT), lambda i, j: (j, 0)),
        ],
        out_specs=pl.BlockSpec((_BD, _BD), lambda i, j: (i, j)),
        out_shape=jax.ShapeDtypeStruct((N_NODES, N_NODES), jnp.float32),
        compiler_params=pltpu.CompilerParams(
            dimension_semantics=("parallel", "parallel")),
    )(z, z)


def kernel(x, edge_index, W1, b1, W_mu, b_mu, W_ls, b_ls, eps):
    src = edge_index[0].astype(jnp.int32)
    dst = edge_index[1].astype(jnp.int32)

    n_extra = E_PAD - src.shape[0]
    ar = jnp.arange(n_extra, dtype=jnp.int32)
    pad_src = (ar * 131) % N_NODES
    pad_dst = N_NODES + (ar % 16)
    src3 = jnp.concatenate([src, pad_src]).reshape(16, NB, BB)
    dst3 = jnp.concatenate([dst, pad_dst]).reshape(16, NB, BB)

    _hist_kernel, _conv_kernel = _sc_kernels()
    partials = _hist_kernel(dst3)
    dis_row = _degree_kernel(partials)
    dis_col = dis_row.reshape(N_PAD, 1)[:N_NODES]

    hsA1, hsB1 = _m1_kernel(x, W1, dis_col)
    aggA1, aggB1 = _conv_kernel(hsA1, hsB1, src3, dst3)
    Wcat = jnp.concatenate([W_mu, W_ls], axis=1)
    hsA2, hsB2 = _e1_kernel(aggA1[:N_NODES], aggB1[:N_NODES], hsA1, hsB1,
                            dis_col, b1.reshape(1, -1), Wcat)
    aggA2, aggB2 = _conv_kernel(hsA2, hsB2, src3, dst3)
    mu, logstd, z = _e2_kernel(aggA2[:N_NODES], aggB2[:N_NODES], hsA2, hsB2,
                               dis_col, b_mu.reshape(1, -1),
                               b_ls.reshape(1, -1), eps)
    adj = _decoder_kernel(z)
    return (adj, mu, logstd)


# speedup vs baseline: 10.7448x; 1.3059x over previous
import dataclasses
import functools

import jax
import jax.numpy as jnp
from jax import lax
from jax.experimental import pallas as pl
from jax.experimental.pallas import tpu as pltpu
from jax.experimental.pallas import tpu_sc as plsc

N_NODES = 10000
D_IN = 256
D_HID = 256
D_LAT = 128

N_PAD = 10112
E_PAD = 163840
NB = 80
BB = 128
ROWS_PER_SUB = N_PAD // 16


def _hist_body(dst3, out, dst_v, hist):
    c = lax.axis_index("c")
    s = lax.axis_index("s")
    wid = c * 16 + s
    pltpu.sync_copy(dst3.at[s], dst_v)

    @pl.loop(0, N_PAD, step=16)
    def _zero(i):
        hist[pl.ds(i, 16)] = jnp.zeros((16,), jnp.float32)

    ones = jnp.ones((16,), jnp.float32)

    @pl.loop(0, NB // 2)
    def _row(j):
        @pl.loop(0, BB, step=16)
        def _vec(k):
            idx = dst_v[j + c * (NB // 2), pl.ds(k, 16)]
            plsc.addupdate_scatter(hist, [idx], ones)

    pltpu.sync_copy(hist, out.at[wid])


def _conv_body(hsA, hsB, src3, dst3, outA, outB, src_v, dst_v, rows0, rows1,
               acc, gsem, ssem):
    c = lax.axis_index("c")
    s = lax.axis_index("s")
    base = s * ROWS_PER_SUB

    @pl.loop(0, BB)
    def _zr(i):
        @pl.loop(0, 128, step=16)
        def _zc(j):
            rows0[i, pl.ds(j, 16)] = jnp.zeros((16,), jnp.float32)

    @pl.loop(0, ROWS_PER_SUB // BB)
    def _za(k):
        pltpu.sync_copy(rows0, acc.at[pl.ds(base + k * BB, BB)])

    pltpu.sync_copy(
        rows0.at[pl.ds(0, ROWS_PER_SUB % BB)],
        acc.at[pl.ds(base + (ROWS_PER_SUB // BB) * BB, ROWS_PER_SUB % BB)],
    )

    pltpu.sync_copy(src3.at[s], src_v)
    pltpu.sync_copy(dst3.at[s], dst_v)
    plsc.subcore_barrier()

    def work(hs_ref):
        @pl.loop(0, NB)
        def _batch(j):
            pltpu.sync_copy(hs_ref.at[src_v.at[j]], rows0)
            pltpu.sync_copy(rows0, acc.at[dst_v.at[j]], add=True)

    @pl.when(c == 0)
    def _w0():
        work(hsA)

    @pl.when(c == 1)
    def _w1():
        work(hsB)

    plsc.subcore_barrier()

    @pl.when(c == 0)
    def _o0():
        pltpu.sync_copy(acc.at[pl.ds(base, ROWS_PER_SUB)],
                        outA.at[pl.ds(base, ROWS_PER_SUB)])

    @pl.when(c == 1)
    def _o1():
        pltpu.sync_copy(acc.at[pl.ds(base, ROWS_PER_SUB)],
                        outB.at[pl.ds(base, ROWS_PER_SUB)])


@functools.cache
def _sc_kernels():
    mesh = plsc.VectorSubcoreMesh(core_axis_name="c", subcore_axis_name="s")
    cp = pltpu.CompilerParams()
    if "needs_layout_passes" in pltpu.CompilerParams.__dataclass_fields__:
        cp = dataclasses.replace(cp, needs_layout_passes=False)
    hist = functools.partial(
        pl.kernel,
        out_type=jax.ShapeDtypeStruct((32, N_PAD), jnp.float32),
        mesh=mesh,
        compiler_params=cp,
        scratch_types=[
            pltpu.VMEM((NB, BB), jnp.int32),
            pltpu.VMEM((N_PAD,), jnp.float32),
        ],
    )(_hist_body)
    conv = functools.partial(
        pl.kernel,
        out_type=[
            jax.ShapeDtypeStruct((N_PAD, 128), jnp.float32),
            jax.ShapeDtypeStruct((N_PAD, 128), jnp.float32),
        ],
        mesh=mesh,
        scratch_types=[
            pltpu.VMEM((NB, BB), jnp.int32),
            pltpu.VMEM((NB, BB), jnp.int32),
            pltpu.VMEM((BB, 128), jnp.float32),
            pltpu.VMEM((BB, 128), jnp.float32),
            pltpu.VMEM_SHARED((N_PAD, 128), jnp.float32),
            pltpu.SemaphoreType.DMA,
            pltpu.SemaphoreType.DMA,
        ],
    )(_conv_body)
    return hist, conv


_BM = 1000
_HIGH = jax.lax.Precision.HIGHEST


def _deg_body(p_ref, dis_ref):
    deg = jnp.sum(p_ref[...], axis=0, keepdims=True) + 1.0
    dis_ref[...] = lax.rsqrt(deg)


def _degree_kernel(partials):
    return pl.pallas_call(
        _deg_body,
        out_shape=jax.ShapeDtypeStruct((1, N_PAD), jnp.float32),
    )(partials)


def _m1_body(x_ref, w_ref, dis_ref, a_ref, b_ref):
    hs = jax.lax.dot_general(x_ref[...], w_ref[...], (((1,), (0,)), ((), ())),
                             precision=_HIGH,
                             preferred_element_type=jnp.float32)
    hs = hs * dis_ref[...]
    a_ref[...] = hs[:, :128]
    b_ref[...] = hs[:, 128:]


def _m1_kernel(x, W1, dis_col):
    return pl.pallas_call(
        _m1_body,
        grid=(N_NODES // _BM,),
        in_specs=[
            pl.BlockSpec((_BM, D_IN), lambda i: (i, 0)),
            pl.BlockSpec((D_IN, D_HID), lambda i: (0, 0)),
            pl.BlockSpec((_BM, 1), lambda i: (i, 0)),
        ],
        out_specs=[
            pl.BlockSpec((_BM, 128), lambda i: (i, 0)),
            pl.BlockSpec((_BM, 128), lambda i: (i, 0)),
        ],
        out_shape=[
            jax.ShapeDtypeStruct((N_NODES, 128), jnp.float32),
            jax.ShapeDtypeStruct((N_NODES, 128), jnp.float32),
        ],
    )(x, W1, dis_col)


def _e1_body(aggA_ref, aggB_ref, hsA_ref, hsB_ref, dis_ref, b1_ref, w_ref,
             a_ref, b_ref):
    dis = dis_ref[...]
    hidden = jnp.concatenate(
        [(aggA_ref[...] + hsA_ref[...]) * dis,
         (aggB_ref[...] + hsB_ref[...]) * dis], axis=1)
    hidden = jnp.maximum(hidden + b1_ref[...], 0.0)
    hs2 = jax.lax.dot_general(hidden, w_ref[...], (((1,), (0,)), ((), ())),
                              precision=_HIGH,
                              preferred_element_type=jnp.float32)
    hs2 = hs2 * dis
    a_ref[...] = hs2[:, :128]
    b_ref[...] = hs2[:, 128:]


def _e1_kernel(aggA, aggB, hsA, hsB, dis_col, b1_row, Wcat):
    blk = lambda i: (i, 0)
    return pl.pallas_call(
        _e1_body,
        grid=(N_NODES // _BM,),
        in_specs=[
            pl.BlockSpec((_BM, 128), blk),
            pl.BlockSpec((_BM, 128), blk),
            pl.BlockSpec((_BM, 128), blk),
            pl.BlockSpec((_BM, 128), blk),
            pl.BlockSpec((_BM, 1), blk),
            pl.BlockSpec((1, D_HID), lambda i: (0, 0)),
            pl.BlockSpec((D_HID, 2 * D_LAT), lambda i: (0, 0)),
        ],
        out_specs=[
            pl.BlockSpec((_BM, 128), blk),
            pl.BlockSpec((_BM, 128), blk),
        ],
        out_shape=[
            jax.ShapeDtypeStruct((N_NODES, 128), jnp.float32),
            jax.ShapeDtypeStruct((N_NODES, 128), jnp.float32),
        ],
    )(aggA, aggB, hsA, hsB, dis_col, b1_row, Wcat)


def _e2_body(aggA_ref, aggB_ref, hsA_ref, hsB_ref, dis_ref, bmu_ref, bls_ref,
             eps_ref, mu_ref, ls_ref, z_ref):
    dis = dis_ref[...]
    mu = (aggA_ref[...] + hsA_ref[...]) * dis + bmu_ref[...]
    ls = (aggB_ref[...] + hsB_ref[...]) * dis + bls_ref[...]
    mu_ref[...] = mu
    ls_ref[...] = ls
    z_ref[...] = mu + eps_ref[...] * jnp.exp(ls)


def _e2_kernel(aggA, aggB, hsA, hsB, dis_col, bmu_row, bls_row, eps):
    blk = lambda i: (i, 0)
    return pl.pallas_call(
        _e2_body,
        grid=(N_NODES // _BM,),
        in_specs=[
            pl.BlockSpec((_BM, 128), blk),
            pl.BlockSpec((_BM, 128), blk),
            pl.BlockSpec((_BM, 128), blk),
            pl.BlockSpec((_BM, 128), blk),
            pl.BlockSpec((_BM, 1), blk),
            pl.BlockSpec((1, D_LAT), lambda i: (0, 0)),
            pl.BlockSpec((1, D_LAT), lambda i: (0, 0)),
            pl.BlockSpec((_BM, D_LAT), blk),
        ],
        out_specs=[pl.BlockSpec((_BM, D_LAT), blk)] * 3,
        out_shape=[jax.ShapeDtypeStruct((N_NODES, D_LAT), jnp.float32)] * 3,
    )(aggA, aggB, hsA, hsB, dis_col, bmu_row, bls_row, eps)


def _dec_body(zi_ref, zj_ref, o_ref):
    o_ref[...] = jax.lax.dot_general(
        zi_ref[...], zj_ref[...], (((1,), (1,)), ((), ())),
        precision=jax.lax.Precision.DEFAULT, preferred_element_type=jnp.float32)


_BD = 1024


def _decoder_kernel(z):
    nblk = (N_NODES + _BD - 1) // _BD
    return pl.pallas_call(
        _dec_body,
        grid=(nblk, nblk),
        in_specs=[
            pl.BlockSpec((_BD, D_LAT), lambda i, j: (i, 0)),
            pl.BlockSpec((_BD, D_LAT), lambda i, j: (j, 0)),
        ],
        out_specs=pl.BlockSpec((_BD, _BD), lambda i, j: (i, j)),
        out_shape=jax.ShapeDtypeStruct((N_NODES, N_NODES), jnp.float32),
        compiler_params=pltpu.CompilerParams(
            dimension_semantics=("parallel", "parallel")),
    )(z, z)


def kernel(x, edge_index, W1, b1, W_mu, b_mu, W_ls, b_ls, eps):
    src = edge_index[0].astype(jnp.int32)
    dst = edge_index[1].astype(jnp.int32)

    n_extra = E_PAD - src.shape[0]
    ar = jnp.arange(n_extra, dtype=jnp.int32)
    pad_src = (ar * 131) % N_NODES
    pad_dst = N_NODES + (ar % 16)
    src3 = jnp.concatenate([src, pad_src]).reshape(16, NB, BB)
    dst3 = jnp.concatenate([dst, pad_dst]).reshape(16, NB, BB)

    _hist_kernel, _conv_kernel = _sc_kernels()
    partials = _hist_kernel(dst3)
    dis_row = _degree_kernel(partials)
    dis_col = dis_row.reshape(N_PAD, 1)[:N_NODES]

    hsA1, hsB1 = _m1_kernel(x, W1, dis_col)
    aggA1, aggB1 = _conv_kernel(hsA1, hsB1, src3, dst3)
    Wcat = jnp.concatenate([W_mu, W_ls], axis=1)
    hsA2, hsB2 = _e1_kernel(aggA1[:N_NODES], aggB1[:N_NODES], hsA1, hsB1,
                            dis_col, b1.reshape(1, -1), Wcat)
    aggA2, aggB2 = _conv_kernel(hsA2, hsB2, src3, dst3)
    mu, logstd, z = _e2_kernel(aggA2[:N_NODES], aggB2[:N_NODES], hsA2, hsB2,
                               dis_col, b_mu.reshape(1, -1),
                               b_ls.reshape(1, -1), eps)
    adj = _decoder_kernel(z)
    return (adj, mu, logstd)


# speedup vs baseline: 10.8672x; 1.0114x over previous
import dataclasses
import functools

import jax
import jax.numpy as jnp
from jax import lax
from jax.experimental import pallas as pl
from jax.experimental.pallas import tpu as pltpu
from jax.experimental.pallas import tpu_sc as plsc

N_NODES = 10000
D_IN = 256
D_HID = 256
D_LAT = 128

N_PAD = 10112
E_PAD = 163840
NB = 80
BB = 128
ROWS_PER_SUB = N_PAD // 16


def _hist_body(dst3, out, dst_v, hist):
    c = lax.axis_index("c")
    s = lax.axis_index("s")
    wid = c * 16 + s
    pltpu.sync_copy(dst3.at[s], dst_v)

    @pl.loop(0, N_PAD, step=16)
    def _zero(i):
        hist[pl.ds(i, 16)] = jnp.zeros((16,), jnp.float32)

    ones = jnp.ones((16,), jnp.float32)

    @pl.loop(0, NB // 2)
    def _row(j):
        @pl.loop(0, BB, step=16)
        def _vec(k):
            idx = dst_v[j + c * (NB // 2), pl.ds(k, 16)]
            plsc.addupdate_scatter(hist, [idx], ones)

    pltpu.sync_copy(hist, out.at[wid])


def _conv_body(hsA, hsB, src3, dst3, outA, outB, src_v, dst_v, rows0, rows1,
               acc, gsem, ssem):
    c = lax.axis_index("c")
    s = lax.axis_index("s")
    base = s * ROWS_PER_SUB

    @pl.loop(0, BB)
    def _zr(i):
        @pl.loop(0, 128, step=16)
        def _zc(j):
            rows0[i, pl.ds(j, 16)] = jnp.zeros((16,), jnp.float32)

    @pl.loop(0, ROWS_PER_SUB // BB)
    def _za(k):
        pltpu.sync_copy(rows0, acc.at[pl.ds(base + k * BB, BB)])

    pltpu.sync_copy(
        rows0.at[pl.ds(0, ROWS_PER_SUB % BB)],
        acc.at[pl.ds(base + (ROWS_PER_SUB // BB) * BB, ROWS_PER_SUB % BB)],
    )

    pltpu.sync_copy(src3.at[s], src_v)
    pltpu.sync_copy(dst3.at[s], dst_v)
    plsc.subcore_barrier()

    def work(hs_ref):
        @pl.loop(0, NB)
        def _batch(j):
            pltpu.sync_copy(hs_ref.at[src_v.at[j]], rows0)
            pltpu.sync_copy(rows0, acc.at[dst_v.at[j]], add=True)

    @pl.when(c == 0)
    def _w0():
        work(hsA)

    @pl.when(c == 1)
    def _w1():
        work(hsB)

    plsc.subcore_barrier()

    @pl.when(c == 0)
    def _o0():
        pltpu.sync_copy(acc.at[pl.ds(base, ROWS_PER_SUB)],
                        outA.at[pl.ds(base, ROWS_PER_SUB)])

    @pl.when(c == 1)
    def _o1():
        pltpu.sync_copy(acc.at[pl.ds(base, ROWS_PER_SUB)],
                        outB.at[pl.ds(base, ROWS_PER_SUB)])


@functools.cache
def _sc_kernels():
    mesh = plsc.VectorSubcoreMesh(core_axis_name="c", subcore_axis_name="s")
    cp = pltpu.CompilerParams()
    if "needs_layout_passes" in pltpu.CompilerParams.__dataclass_fields__:
        cp = dataclasses.replace(cp, needs_layout_passes=False)
    hist = functools.partial(
        pl.kernel,
        out_type=jax.ShapeDtypeStruct((32, N_PAD), jnp.float32),
        mesh=mesh,
        compiler_params=cp,
        scratch_types=[
            pltpu.VMEM((NB, BB), jnp.int32),
            pltpu.VMEM((N_PAD,), jnp.float32),
        ],
    )(_hist_body)
    conv = functools.partial(
        pl.kernel,
        out_type=[
            jax.ShapeDtypeStruct((N_PAD, 128), jnp.float32),
            jax.ShapeDtypeStruct((N_PAD, 128), jnp.float32),
        ],
        mesh=mesh,
        scratch_types=[
            pltpu.VMEM((NB, BB), jnp.int32),
            pltpu.VMEM((NB, BB), jnp.int32),
            pltpu.VMEM((BB, 128), jnp.float32),
            pltpu.VMEM((BB, 128), jnp.float32),
            pltpu.VMEM_SHARED((N_PAD, 128), jnp.float32),
            pltpu.SemaphoreType.DMA,
            pltpu.SemaphoreType.DMA,
        ],
    )(_conv_body)
    return hist, conv


_BM = 1000
_HIGH = jax.lax.Precision.DEFAULT


def _deg_body(p_ref, dis_ref):
    deg = jnp.sum(p_ref[...], axis=0, keepdims=True) + 1.0
    dis_ref[...] = lax.rsqrt(deg)


def _degree_kernel(partials):
    return pl.pallas_call(
        _deg_body,
        out_shape=jax.ShapeDtypeStruct((1, N_PAD), jnp.float32),
    )(partials)


def _m1_body(x_ref, w_ref, dis_ref, a_ref, b_ref):
    hs = jax.lax.dot_general(x_ref[...], w_ref[...], (((1,), (0,)), ((), ())),
                             precision=_HIGH,
                             preferred_element_type=jnp.float32)
    hs = hs * dis_ref[...]
    a_ref[...] = hs[:, :128]
    b_ref[...] = hs[:, 128:]


def _m1_kernel(x, W1, dis_col):
    return pl.pallas_call(
        _m1_body,
        grid=(N_NODES // _BM,),
        in_specs=[
            pl.BlockSpec((_BM, D_IN), lambda i: (i, 0)),
            pl.BlockSpec((D_IN, D_HID), lambda i: (0, 0)),
            pl.BlockSpec((_BM, 1), lambda i: (i, 0)),
        ],
        out_specs=[
            pl.BlockSpec((_BM, 128), lambda i: (i, 0)),
            pl.BlockSpec((_BM, 128), lambda i: (i, 0)),
        ],
        out_shape=[
            jax.ShapeDtypeStruct((N_NODES, 128), jnp.float32),
            jax.ShapeDtypeStruct((N_NODES, 128), jnp.float32),
        ],
    )(x, W1, dis_col)


def _e1_body(aggA_ref, aggB_ref, hsA_ref, hsB_ref, dis_ref, b1_ref, w_ref,
             a_ref, b_ref):
    dis = dis_ref[...]
    hidden = jnp.concatenate(
        [(aggA_ref[...] + hsA_ref[...]) * dis,
         (aggB_ref[...] + hsB_ref[...]) * dis], axis=1)
    hidden = jnp.maximum(hidden + b1_ref[...], 0.0)
    hs2 = jax.lax.dot_general(hidden, w_ref[...], (((1,), (0,)), ((), ())),
                              precision=_HIGH,
                              preferred_element_type=jnp.float32)
    hs2 = hs2 * dis
    a_ref[...] = hs2[:, :128]
    b_ref[...] = hs2[:, 128:]


def _e1_kernel(aggA, aggB, hsA, hsB, dis_col, b1_row, Wcat):
    blk = lambda i: (i, 0)
    return pl.pallas_call(
        _e1_body,
        grid=(N_NODES // _BM,),
        in_specs=[
            pl.BlockSpec((_BM, 128), blk),
            pl.BlockSpec((_BM, 128), blk),
            pl.BlockSpec((_BM, 128), blk),
            pl.BlockSpec((_BM, 128), blk),
            pl.BlockSpec((_BM, 1), blk),
            pl.BlockSpec((1, D_HID), lambda i: (0, 0)),
            pl.BlockSpec((D_HID, 2 * D_LAT), lambda i: (0, 0)),
        ],
        out_specs=[
            pl.BlockSpec((_BM, 128), blk),
            pl.BlockSpec((_BM, 128), blk),
        ],
        out_shape=[
            jax.ShapeDtypeStruct((N_NODES, 128), jnp.float32),
            jax.ShapeDtypeStruct((N_NODES, 128), jnp.float32),
        ],
    )(aggA, aggB, hsA, hsB, dis_col, b1_row, Wcat)


def _e2_body(aggA_ref, aggB_ref, hsA_ref, hsB_ref, dis_ref, bmu_ref, bls_ref,
             eps_ref, mu_ref, ls_ref, z_ref):
    dis = dis_ref[...]
    mu = (aggA_ref[...] + hsA_ref[...]) * dis + bmu_ref[...]
    ls = (aggB_ref[...] + hsB_ref[...]) * dis + bls_ref[...]
    mu_ref[...] = mu
    ls_ref[...] = ls
    z_ref[...] = mu + eps_ref[...] * jnp.exp(ls)


def _e2_kernel(aggA, aggB, hsA, hsB, dis_col, bmu_row, bls_row, eps):
    blk = lambda i: (i, 0)
    return pl.pallas_call(
        _e2_body,
        grid=(N_NODES // _BM,),
        in_specs=[
            pl.BlockSpec((_BM, 128), blk),
            pl.BlockSpec((_BM, 128), blk),
            pl.BlockSpec((_BM, 128), blk),
            pl.BlockSpec((_BM, 128), blk),
            pl.BlockSpec((_BM, 1), blk),
            pl.BlockSpec((1, D_LAT), lambda i: (0, 0)),
            pl.BlockSpec((1, D_LAT), lambda i: (0, 0)),
            pl.BlockSpec((_BM, D_LAT), blk),
        ],
        out_specs=[pl.BlockSpec((_BM, D_LAT), blk)] * 3,
        out_shape=[jax.ShapeDtypeStruct((N_NODES, D_LAT), jnp.float32)] * 3,
    )(aggA, aggB, hsA, hsB, dis_col, bmu_row, bls_row, eps)


def _dec_body(zi_ref, zj_ref, o_ref):
    o_ref[...] = jax.lax.dot_general(
        zi_ref[...], zj_ref[...], (((1,), (1,)), ((), ())),
        precision=jax.lax.Precision.DEFAULT, preferred_element_type=jnp.float32)


_BD = 1024


def _decoder_kernel(z):
    nblk = (N_NODES + _BD - 1) // _BD
    return pl.pallas_call(
        _dec_body,
        grid=(nblk, nblk),
        in_specs=[
            pl.BlockSpec((_BD, D_LAT), lambda i, j: (i, 0)),
            pl.BlockSpec((_BD, D_LAT), lambda i, j: (j, 0)),
        ],
        out_specs=pl.BlockSpec((_BD, _BD), lambda i, j: (i, j)),
        out_shape=jax.ShapeDtypeStruct((N_NODES, N_NODES), jnp.float32),
        compiler_params=pltpu.CompilerParams(
            dimension_semantics=("parallel", "parallel")),
    )(z, z)


def kernel(x, edge_index, W1, b1, W_mu, b_mu, W_ls, b_ls, eps):
    src = edge_index[0].astype(jnp.int32)
    dst = edge_index[1].astype(jnp.int32)

    n_extra = E_PAD - src.shape[0]
    ar = jnp.arange(n_extra, dtype=jnp.int32)
    pad_src = (ar * 131) % N_NODES
    pad_dst = N_NODES + (ar % 16)
    src3 = jnp.concatenate([src, pad_src]).reshape(16, NB, BB)
    dst3 = jnp.concatenate([dst, pad_dst]).reshape(16, NB, BB)

    _hist_kernel, _conv_kernel = _sc_kernels()
    partials = _hist_kernel(dst3)
    dis_row = _degree_kernel(partials)
    dis_col = dis_row.reshape(N_PAD, 1)[:N_NODES]

    hsA1, hsB1 = _m1_kernel(x, W1, dis_col)
    aggA1, aggB1 = _conv_kernel(hsA1, hsB1, src3, dst3)
    Wcat = jnp.concatenate([W_mu, W_ls], axis=1)
    hsA2, hsB2 = _e1_kernel(aggA1[:N_NODES], aggB1[:N_NODES], hsA1, hsB1,
                            dis_col, b1.reshape(1, -1), Wcat)
    aggA2, aggB2 = _conv_kernel(hsA2, hsB2, src3, dst3)
    mu, logstd, z = _e2_kernel(aggA2[:N_NODES], aggB2[:N_NODES], hsA2, hsB2,
                               dis_col, b_mu.reshape(1, -1),
                               b_ls.reshape(1, -1), eps)
    adj = _decoder_kernel(z)
    return (adj, mu, logstd)


# speedup vs baseline: 11.8184x; 1.0875x over previous
import dataclasses
import functools

import jax
import jax.numpy as jnp
from jax import lax
from jax.experimental import pallas as pl
from jax.experimental.pallas import tpu as pltpu
from jax.experimental.pallas import tpu_sc as plsc

N_NODES = 10000
D_IN = 256
D_HID = 256
D_LAT = 128

N_PAD = 10112
E_PAD = 163840
NB = 80
BB = 128
ROWS_PER_SUB = N_PAD // 16


def _hist_body(dst3, out, dst_v, hist):
    c = lax.axis_index("c")
    s = lax.axis_index("s")
    wid = c * 16 + s
    pltpu.sync_copy(dst3.at[s], dst_v)

    @pl.loop(0, N_PAD, step=16)
    def _zero(i):
        hist[pl.ds(i, 16)] = jnp.zeros((16,), jnp.float32)

    ones = jnp.ones((16,), jnp.float32)

    @pl.loop(0, NB // 2)
    def _row(j):
        @pl.loop(0, BB, step=16)
        def _vec(k):
            idx = dst_v[j + c * (NB // 2), pl.ds(k, 16)]
            plsc.addupdate_scatter(hist, [idx], ones)

    pltpu.sync_copy(hist, out.at[wid])


def _conv_body(hsA, hsB, src3, dst3, outA, outB, src_v, dst_v, rows0, rows1,
               acc, sem0, sem1):
    c = lax.axis_index("c")
    s = lax.axis_index("s")
    base = s * ROWS_PER_SUB

    @pl.loop(0, BB)
    def _zr(i):
        @pl.loop(0, 128, step=16)
        def _zc(j):
            rows0[i, pl.ds(j, 16)] = jnp.zeros((16,), jnp.float32)

    @pl.loop(0, ROWS_PER_SUB // BB)
    def _za(k):
        pltpu.sync_copy(rows0, acc.at[pl.ds(base + k * BB, BB)])

    pltpu.sync_copy(
        rows0.at[pl.ds(0, ROWS_PER_SUB % BB)],
        acc.at[pl.ds(base + (ROWS_PER_SUB // BB) * BB, ROWS_PER_SUB % BB)],
    )

    plsc.subcore_barrier()

    def work(hs_ref):
        @pl.loop(0, 2)
        def _half(h):
            pltpu.sync_copy(src3.at[s].at[pl.ds(h * (NB // 2), NB // 2)],
                            src_v)
            pltpu.sync_copy(dst3.at[s].at[pl.ds(h * (NB // 2), NB // 2)],
                            dst_v)

            @pl.loop(0, NB // 4)
            def _pair(jh):
                j = jh * 2
                g0 = pltpu.async_copy(hs_ref.at[src_v.at[j]], rows0, sem0)
                g1 = pltpu.async_copy(hs_ref.at[src_v.at[j + 1]], rows1, sem1)
                g0.wait()
                pltpu.sync_copy(rows0, acc.at[dst_v.at[j]], add=True)
                g1.wait()
                pltpu.sync_copy(rows1, acc.at[dst_v.at[j + 1]], add=True)

    @pl.when(c == 0)
    def _w0():
        work(hsA)

    @pl.when(c == 1)
    def _w1():
        work(hsB)

    plsc.subcore_barrier()

    @pl.when(c == 0)
    def _o0():
        pltpu.sync_copy(acc.at[pl.ds(base, ROWS_PER_SUB)],
                        outA.at[pl.ds(base, ROWS_PER_SUB)])

    @pl.when(c == 1)
    def _o1():
        pltpu.sync_copy(acc.at[pl.ds(base, ROWS_PER_SUB)],
                        outB.at[pl.ds(base, ROWS_PER_SUB)])


@functools.cache
def _sc_kernels():
    mesh = plsc.VectorSubcoreMesh(core_axis_name="c", subcore_axis_name="s")
    cp = pltpu.CompilerParams()
    if "needs_layout_passes" in pltpu.CompilerParams.__dataclass_fields__:
        cp = dataclasses.replace(cp, needs_layout_passes=False)
    hist = functools.partial(
        pl.kernel,
        out_type=jax.ShapeDtypeStruct((32, N_PAD), jnp.float32),
        mesh=mesh,
        compiler_params=cp,
        scratch_types=[
            pltpu.VMEM((NB, BB), jnp.int32),
            pltpu.VMEM((N_PAD,), jnp.float32),
        ],
    )(_hist_body)
    conv = functools.partial(
        pl.kernel,
        out_type=[
            jax.ShapeDtypeStruct((N_PAD, 128), jnp.float32),
            jax.ShapeDtypeStruct((N_PAD, 128), jnp.float32),
        ],
        mesh=mesh,
        scratch_types=[
            pltpu.VMEM((NB // 2, BB), jnp.int32),
            pltpu.VMEM((NB // 2, BB), jnp.int32),
            pltpu.VMEM((BB, 128), jnp.float32),
            pltpu.VMEM((BB, 128), jnp.float32),
            pltpu.VMEM_SHARED((N_PAD, 128), jnp.float32),
            pltpu.SemaphoreType.DMA,
            pltpu.SemaphoreType.DMA,
        ],
    )(_conv_body)
    return hist, conv


_BM = 1000
_HIGH = jax.lax.Precision.DEFAULT


def _deg_body(p_ref, dis_ref):
    deg = jnp.sum(p_ref[...], axis=0, keepdims=True) + 1.0
    dis_ref[...] = lax.rsqrt(deg)


def _degree_kernel(partials):
    return pl.pallas_call(
        _deg_body,
        out_shape=jax.ShapeDtypeStruct((1, N_PAD), jnp.float32),
    )(partials)


def _m1_body(x_ref, w_ref, dis_ref, a_ref, b_ref):
    hs = jax.lax.dot_general(x_ref[...], w_ref[...], (((1,), (0,)), ((), ())),
                             precision=_HIGH,
                             preferred_element_type=jnp.float32)
    hs = hs * dis_ref[...]
    a_ref[...] = hs[:, :128]
    b_ref[...] = hs[:, 128:]


def _m1_kernel(x, W1, dis_col):
    return pl.pallas_call(
        _m1_body,
        grid=(N_NODES // _BM,),
        in_specs=[
            pl.BlockSpec((_BM, D_IN), lambda i: (i, 0)),
            pl.BlockSpec((D_IN, D_HID), lambda i: (0, 0)),
            pl.BlockSpec((_BM, 1), lambda i: (i, 0)),
        ],
        out_specs=[
            pl.BlockSpec((_BM, 128), lambda i: (i, 0)),
            pl.BlockSpec((_BM, 128), lambda i: (i, 0)),
        ],
        out_shape=[
            jax.ShapeDtypeStruct((N_NODES, 128), jnp.float32),
            jax.ShapeDtypeStruct((N_NODES, 128), jnp.float32),
        ],
    )(x, W1, dis_col)


def _e1_body(aggA_ref, aggB_ref, hsA_ref, hsB_ref, dis_ref, b1_ref, w_ref,
             a_ref, b_ref):
    dis = dis_ref[...]
    hidden = jnp.concatenate(
        [(aggA_ref[...] + hsA_ref[...]) * dis,
         (aggB_ref[...] + hsB_ref[...]) * dis], axis=1)
    hidden = jnp.maximum(hidden + b1_ref[...], 0.0)
    hs2 = jax.lax.dot_general(hidden, w_ref[...], (((1,), (0,)), ((), ())),
                              precision=_HIGH,
                              preferred_element_type=jnp.float32)
    hs2 = hs2 * dis
    a_ref[...] = hs2[:, :128]
    b_ref[...] = hs2[:, 128:]


def _e1_kernel(aggA, aggB, hsA, hsB, dis_col, b1_row, Wcat):
    blk = lambda i: (i, 0)
    return pl.pallas_call(
        _e1_body,
        grid=(N_NODES // _BM,),
        in_specs=[
            pl.BlockSpec((_BM, 128), blk),
            pl.BlockSpec((_BM, 128), blk),
            pl.BlockSpec((_BM, 128), blk),
            pl.BlockSpec((_BM, 128), blk),
            pl.BlockSpec((_BM, 1), blk),
            pl.BlockSpec((1, D_HID), lambda i: (0, 0)),
            pl.BlockSpec((D_HID, 2 * D_LAT), lambda i: (0, 0)),
        ],
        out_specs=[
            pl.BlockSpec((_BM, 128), blk),
            pl.BlockSpec((_BM, 128), blk),
        ],
        out_shape=[
            jax.ShapeDtypeStruct((N_NODES, 128), jnp.float32),
            jax.ShapeDtypeStruct((N_NODES, 128), jnp.float32),
        ],
    )(aggA, aggB, hsA, hsB, dis_col, b1_row, Wcat)


def _e2_body(aggA_ref, aggB_ref, hsA_ref, hsB_ref, dis_ref, bmu_ref, bls_ref,
             eps_ref, mu_ref, ls_ref, z_ref):
    dis = dis_ref[...]
    mu = (aggA_ref[...] + hsA_ref[...]) * dis + bmu_ref[...]
    ls = (aggB_ref[...] + hsB_ref[...]) * dis + bls_ref[...]
    mu_ref[...] = mu
    ls_ref[...] = ls
    z_ref[...] = mu + eps_ref[...] * jnp.exp(ls)


def _e2_kernel(aggA, aggB, hsA, hsB, dis_col, bmu_row, bls_row, eps):
    blk = lambda i: (i, 0)
    return pl.pallas_call(
        _e2_body,
        grid=(N_NODES // _BM,),
        in_specs=[
            pl.BlockSpec((_BM, 128), blk),
            pl.BlockSpec((_BM, 128), blk),
            pl.BlockSpec((_BM, 128), blk),
            pl.BlockSpec((_BM, 128), blk),
            pl.BlockSpec((_BM, 1), blk),
            pl.BlockSpec((1, D_LAT), lambda i: (0, 0)),
            pl.BlockSpec((1, D_LAT), lambda i: (0, 0)),
            pl.BlockSpec((_BM, D_LAT), blk),
        ],
        out_specs=[pl.BlockSpec((_BM, D_LAT), blk)] * 3,
        out_shape=[jax.ShapeDtypeStruct((N_NODES, D_LAT), jnp.float32)] * 3,
    )(aggA, aggB, hsA, hsB, dis_col, bmu_row, bls_row, eps)


def _dec_body(zi_ref, zj_ref, o_ref):
    o_ref[...] = jax.lax.dot_general(
        zi_ref[...], zj_ref[...], (((1,), (1,)), ((), ())),
        precision=jax.lax.Precision.DEFAULT, preferred_element_type=jnp.float32)


_BD = 1024


def _decoder_kernel(z):
    nblk = (N_NODES + _BD - 1) // _BD
    return pl.pallas_call(
        _dec_body,
        grid=(nblk, nblk),
        in_specs=[
            pl.BlockSpec((_BD, D_LAT), lambda i, j: (i, 0)),
            pl.BlockSpec((_BD, D_LAT), lambda i, j: (j, 0)),
        ],
        out_specs=pl.BlockSpec((_BD, _BD), lambda i, j: (i, j)),
        out_shape=jax.ShapeDtypeStruct((N_NODES, N_NODES), jnp.float32),
        compiler_params=pltpu.CompilerParams(
            dimension_semantics=("parallel", "parallel")),
    )(z, z)


def kernel(x, edge_index, W1, b1, W_mu, b_mu, W_ls, b_ls, eps):
    src = edge_index[0].astype(jnp.int32)
    dst = edge_index[1].astype(jnp.int32)

    n_extra = E_PAD - src.shape[0]
    ar = jnp.arange(n_extra, dtype=jnp.int32)
    pad_src = (ar * 131) % N_NODES
    pad_dst = N_NODES + (ar % 16)
    src3 = jnp.concatenate([src, pad_src]).reshape(16, NB, BB)
    dst3 = jnp.concatenate([dst, pad_dst]).reshape(16, NB, BB)

    _hist_kernel, _conv_kernel = _sc_kernels()
    partials = _hist_kernel(dst3)
    dis_row = _degree_kernel(partials)
    dis_col = dis_row.reshape(N_PAD, 1)[:N_NODES]

    hsA1, hsB1 = _m1_kernel(x, W1, dis_col)
    aggA1, aggB1 = _conv_kernel(hsA1, hsB1, src3, dst3)
    Wcat = jnp.concatenate([W_mu, W_ls], axis=1)
    hsA2, hsB2 = _e1_kernel(aggA1[:N_NODES], aggB1[:N_NODES], hsA1, hsB1,
                            dis_col, b1.reshape(1, -1), Wcat)
    aggA2, aggB2 = _conv_kernel(hsA2, hsB2, src3, dst3)
    mu, logstd, z = _e2_kernel(aggA2[:N_NODES], aggB2[:N_NODES], hsA2, hsB2,
                               dis_col, b_mu.reshape(1, -1),
                               b_ls.reshape(1, -1), eps)
    adj = _decoder_kernel(z)
    return (adj, mu, logstd)


# speedup vs baseline: 12.1627x; 1.0291x over previous
import dataclasses
import functools

import jax
import jax.numpy as jnp
from jax import lax
from jax.experimental import pallas as pl
from jax.experimental.pallas import tpu as pltpu
from jax.experimental.pallas import tpu_sc as plsc

N_NODES = 10000
D_IN = 256
D_HID = 256
D_LAT = 128

N_PAD = 10112
E_PAD = 163840
NB = 80
BB = 128
ROWS_PER_SUB = N_PAD // 16


def _hist_body(dst3, out, dst_v, hist):
    c = lax.axis_index("c")
    s = lax.axis_index("s")
    wid = c * 16 + s
    pltpu.sync_copy(dst3.at[s], dst_v)

    @pl.loop(0, N_PAD, step=16)
    def _zero(i):
        hist[pl.ds(i, 16)] = jnp.zeros((16,), jnp.float32)

    ones = jnp.ones((16,), jnp.float32)

    @pl.loop(0, NB // 2)
    def _row(j):
        @pl.loop(0, BB, step=16)
        def _vec(k):
            idx = dst_v[j + c * (NB // 2), pl.ds(k, 16)]
            plsc.addupdate_scatter(hist, [idx], ones)

    pltpu.sync_copy(hist, out.at[wid])


def _conv_body(hsA, hsB, src3, dst3, outA, outB, src_v, dst_v, rows0, rows1,
               acc, sem0, sem1):
    c = lax.axis_index("c")
    s = lax.axis_index("s")
    base = s * ROWS_PER_SUB

    @pl.loop(0, BB)
    def _zr(i):
        @pl.loop(0, 128, step=16)
        def _zc(j):
            rows0[i, pl.ds(j, 16)] = jnp.zeros((16,), jnp.float32)

    @pl.loop(0, ROWS_PER_SUB // BB)
    def _za(k):
        pltpu.sync_copy(rows0, acc.at[pl.ds(base + k * BB, BB)])

    pltpu.sync_copy(
        rows0.at[pl.ds(0, ROWS_PER_SUB % BB)],
        acc.at[pl.ds(base + (ROWS_PER_SUB // BB) * BB, ROWS_PER_SUB % BB)],
    )

    plsc.subcore_barrier()

    def work(hs_ref):
        @pl.loop(0, 2)
        def _half(h):
            pltpu.sync_copy(src3.at[s].at[pl.ds(h * (NB // 2), NB // 2)],
                            src_v)
            pltpu.sync_copy(dst3.at[s].at[pl.ds(h * (NB // 2), NB // 2)],
                            dst_v)

            @pl.loop(0, NB // 4)
            def _pair(jh):
                j = jh * 2
                g0 = pltpu.async_copy(hs_ref.at[src_v.at[j]], rows0, sem0)
                g1 = pltpu.async_copy(hs_ref.at[src_v.at[j + 1]], rows1, sem1)
                g0.wait()
                pltpu.sync_copy(rows0, acc.at[dst_v.at[j]], add=True)
                g1.wait()
                pltpu.sync_copy(rows1, acc.at[dst_v.at[j + 1]], add=True)

    @pl.when(c == 0)
    def _w0():
        work(hsA)

    @pl.when(c == 1)
    def _w1():
        work(hsB)

    plsc.subcore_barrier()

    @pl.when(c == 0)
    def _o0():
        pltpu.sync_copy(acc.at[pl.ds(base, ROWS_PER_SUB)],
                        outA.at[pl.ds(base, ROWS_PER_SUB)])

    @pl.when(c == 1)
    def _o1():
        pltpu.sync_copy(acc.at[pl.ds(base, ROWS_PER_SUB)],
                        outB.at[pl.ds(base, ROWS_PER_SUB)])


@functools.cache
def _sc_kernels():
    mesh = plsc.VectorSubcoreMesh(core_axis_name="c", subcore_axis_name="s")
    cp = pltpu.CompilerParams()
    if "needs_layout_passes" in pltpu.CompilerParams.__dataclass_fields__:
        cp = dataclasses.replace(cp, needs_layout_passes=False)
    hist = functools.partial(
        pl.kernel,
        out_type=jax.ShapeDtypeStruct((32, N_PAD), jnp.float32),
        mesh=mesh,
        compiler_params=cp,
        scratch_types=[
            pltpu.VMEM((NB, BB), jnp.int32),
            pltpu.VMEM((N_PAD,), jnp.float32),
        ],
    )(_hist_body)
    conv = functools.partial(
        pl.kernel,
        out_type=[
            jax.ShapeDtypeStruct((N_PAD, 128), jnp.float32),
            jax.ShapeDtypeStruct((N_PAD, 128), jnp.float32),
        ],
        mesh=mesh,
        scratch_types=[
            pltpu.VMEM((NB // 2, BB), jnp.int32),
            pltpu.VMEM((NB // 2, BB), jnp.int32),
            pltpu.VMEM((BB, 128), jnp.float32),
            pltpu.VMEM((BB, 128), jnp.float32),
            pltpu.VMEM_SHARED((N_PAD, 128), jnp.float32),
            pltpu.SemaphoreType.DMA,
            pltpu.SemaphoreType.DMA,
        ],
    )(_conv_body)
    return hist, conv


_BM = 1000
_HIGH = jax.lax.Precision.DEFAULT


def _deg_body(p_ref, dis_ref):
    deg = jnp.sum(p_ref[...], axis=0, keepdims=True) + 1.0
    dis_ref[...] = lax.rsqrt(deg)


def _degree_kernel(partials):
    return pl.pallas_call(
        _deg_body,
        out_shape=jax.ShapeDtypeStruct((1, N_PAD), jnp.float32),
    )(partials)


def _m1_body(x_ref, w_ref, dis_ref, a_ref, b_ref):
    hs = jax.lax.dot_general(x_ref[...], w_ref[...], (((1,), (0,)), ((), ())),
                             precision=_HIGH,
                             preferred_element_type=jnp.float32)
    hs = hs * dis_ref[...]
    a_ref[...] = hs[:, :128]
    b_ref[...] = hs[:, 128:]


def _m1_kernel(x, W1, dis_col):
    return pl.pallas_call(
        _m1_body,
        grid=(N_NODES // _BM,),
        in_specs=[
            pl.BlockSpec((_BM, D_IN), lambda i: (i, 0)),
            pl.BlockSpec((D_IN, D_HID), lambda i: (0, 0)),
            pl.BlockSpec((_BM, 1), lambda i: (i, 0)),
        ],
        out_specs=[
            pl.BlockSpec((_BM, 128), lambda i: (i, 0)),
            pl.BlockSpec((_BM, 128), lambda i: (i, 0)),
        ],
        out_shape=[
            jax.ShapeDtypeStruct((N_NODES, 128), jnp.float32),
            jax.ShapeDtypeStruct((N_NODES, 128), jnp.float32),
        ],
    )(x, W1, dis_col)


def _e1_body(aggA_ref, aggB_ref, hsA_ref, hsB_ref, dis_ref, b1_ref, w_ref,
             a_ref, b_ref):
    dis = dis_ref[...]
    hidden = jnp.concatenate(
        [(aggA_ref[...] + hsA_ref[...]) * dis,
         (aggB_ref[...] + hsB_ref[...]) * dis], axis=1)
    hidden = jnp.maximum(hidden + b1_ref[...], 0.0)
    hs2 = jax.lax.dot_general(hidden, w_ref[...], (((1,), (0,)), ((), ())),
                              precision=_HIGH,
                              preferred_element_type=jnp.float32)
    hs2 = hs2 * dis
    a_ref[...] = hs2[:, :128]
    b_ref[...] = hs2[:, 128:]


def _e1_kernel(aggA, aggB, hsA, hsB, dis_col, b1_row, Wcat):
    blk = lambda i: (i, 0)
    return pl.pallas_call(
        _e1_body,
        grid=(N_NODES // _BM,),
        in_specs=[
            pl.BlockSpec((_BM, 128), blk),
            pl.BlockSpec((_BM, 128), blk),
            pl.BlockSpec((_BM, 128), blk),
            pl.BlockSpec((_BM, 128), blk),
            pl.BlockSpec((_BM, 1), blk),
            pl.BlockSpec((1, D_HID), lambda i: (0, 0)),
            pl.BlockSpec((D_HID, 2 * D_LAT), lambda i: (0, 0)),
        ],
        out_specs=[
            pl.BlockSpec((_BM, 128), blk),
            pl.BlockSpec((_BM, 128), blk),
        ],
        out_shape=[
            jax.ShapeDtypeStruct((N_NODES, 128), jnp.float32),
            jax.ShapeDtypeStruct((N_NODES, 128), jnp.float32),
        ],
    )(aggA, aggB, hsA, hsB, dis_col, b1_row, Wcat)


def _e2_body(aggA_ref, aggB_ref, hsA_ref, hsB_ref, dis_ref, bmu_ref, bls_ref,
             eps_ref, mu_ref, ls_ref, z_ref):
    dis = dis_ref[...]
    mu = (aggA_ref[...] + hsA_ref[...]) * dis + bmu_ref[...]
    ls = (aggB_ref[...] + hsB_ref[...]) * dis + bls_ref[...]
    mu_ref[...] = mu
    ls_ref[...] = ls
    z_ref[...] = mu + eps_ref[...] * jnp.exp(ls)


def _e2_kernel(aggA, aggB, hsA, hsB, dis_col, bmu_row, bls_row, eps):
    blk = lambda i: (i, 0)
    return pl.pallas_call(
        _e2_body,
        grid=(N_NODES // _BM,),
        in_specs=[
            pl.BlockSpec((_BM, 128), blk),
            pl.BlockSpec((_BM, 128), blk),
            pl.BlockSpec((_BM, 128), blk),
            pl.BlockSpec((_BM, 128), blk),
            pl.BlockSpec((_BM, 1), blk),
            pl.BlockSpec((1, D_LAT), lambda i: (0, 0)),
            pl.BlockSpec((1, D_LAT), lambda i: (0, 0)),
            pl.BlockSpec((_BM, D_LAT), blk),
        ],
        out_specs=[pl.BlockSpec((_BM, D_LAT), blk)] * 3,
        out_shape=[jax.ShapeDtypeStruct((N_NODES, D_LAT), jnp.float32)] * 3,
    )(aggA, aggB, hsA, hsB, dis_col, bmu_row, bls_row, eps)


def _dec_body(zi_ref, zj_ref, o_ref):
    o_ref[...] = jax.lax.dot_general(
        zi_ref[...], zj_ref[...], (((1,), (1,)), ((), ())),
        precision=jax.lax.Precision.DEFAULT, preferred_element_type=jnp.float32)


_BD = 1024


def _decoder_kernel(z):
    nblk = (N_NODES + _BD - 1) // _BD
    return pl.pallas_call(
        _dec_body,
        grid=(nblk, nblk),
        in_specs=[
            pl.BlockSpec((_BD, D_LAT), lambda i, j: (i, 0)),
            pl.BlockSpec((_BD, D_LAT), lambda i, j: (j, 0)),
        ],
        out_specs=pl.BlockSpec((_BD, _BD), lambda i, j: (i, j)),
        out_shape=jax.ShapeDtypeStruct((N_NODES, N_NODES), jnp.float32),
        compiler_params=pltpu.CompilerParams(
            dimension_semantics=("parallel", "parallel")),
    )(z, z)


def kernel(x, edge_index, W1, b1, W_mu, b_mu, W_ls, b_ls, eps):
    src = edge_index[0].astype(jnp.int32)
    dst = edge_index[1].astype(jnp.int32)

    n_extra = E_PAD - src.shape[0]
    ar = jnp.arange(n_extra, dtype=jnp.int32)
    pad_src = (ar * 131) % N_NODES
    pad_dst = N_NODES + (ar % 16)
    src3 = jnp.concatenate([src, pad_src]).reshape(16, NB, BB)
    dst3 = jnp.concatenate([dst, pad_dst]).reshape(16, NB, BB)

    _hist_kernel, _conv_kernel = _sc_kernels()
    partials = _hist_kernel(dst3)
    dis_row = _degree_kernel(partials)
    dis_col = dis_row.reshape(N_PAD, 1)[:N_NODES]

    hsA1, hsB1 = _m1_kernel(x, W1, dis_col)
    aggA1, aggB1 = _conv_kernel(hsA1, hsB1, src3, dst3)
    Wcat = jnp.concatenate([W_mu, W_ls], axis=1)
    hsA2, hsB2 = _e1_kernel(aggA1, aggB1, hsA1, hsB1,
                            dis_col, b1.reshape(1, -1), Wcat)
    aggA2, aggB2 = _conv_kernel(hsA2, hsB2, src3, dst3)
    mu, logstd, z = _e2_kernel(aggA2, aggB2, hsA2, hsB2,
                               dis_col, b_mu.reshape(1, -1),
                               b_ls.reshape(1, -1), eps)
    adj = _decoder_kernel(z)
    return (adj, mu, logstd)


# speedup vs baseline: 12.4330x; 1.0222x over previous
import dataclasses
import functools

import jax
import jax.numpy as jnp
from jax import lax
from jax.experimental import pallas as pl
from jax.experimental.pallas import tpu as pltpu
from jax.experimental.pallas import tpu_sc as plsc

N_NODES = 10000
D_IN = 256
D_HID = 256
D_LAT = 128

N_PAD = 10112
E_PAD = 163840
NB = 80
BB = 128
ROWS_PER_SUB = N_PAD // 16


def _hist_body(dst3, out, dst_v, hist):
    c = lax.axis_index("c")
    s = lax.axis_index("s")
    wid = c * 16 + s
    pltpu.sync_copy(dst3.at[s], dst_v)

    @pl.loop(0, N_PAD, step=16)
    def _zero(i):
        hist[pl.ds(i, 16)] = jnp.zeros((16,), jnp.float32)

    ones = jnp.ones((16,), jnp.float32)

    @pl.loop(0, NB // 2)
    def _row(j):
        @pl.loop(0, BB, step=16)
        def _vec(k):
            idx = dst_v[j + c * (NB // 2), pl.ds(k, 16)]
            plsc.addupdate_scatter(hist, [idx], ones)

    pltpu.sync_copy(hist, out.at[wid])


def _conv_body(hsA, hsB, src3, dst3, outA, outB, src_v, dst_v, rows0, rows1,
               acc, sem0, sem1, ssem0, ssem1):
    c = lax.axis_index("c")
    s = lax.axis_index("s")
    base = s * ROWS_PER_SUB

    @pl.loop(0, BB)
    def _zr(i):
        @pl.loop(0, 128, step=16)
        def _zc(j):
            rows0[i, pl.ds(j, 16)] = jnp.zeros((16,), jnp.float32)

    @pl.loop(0, ROWS_PER_SUB // BB)
    def _za(k):
        pltpu.sync_copy(rows0, acc.at[pl.ds(base + k * BB, BB)])

    pltpu.sync_copy(
        rows0.at[pl.ds(0, ROWS_PER_SUB % BB)],
        acc.at[pl.ds(base + (ROWS_PER_SUB // BB) * BB, ROWS_PER_SUB % BB)],
    )

    plsc.subcore_barrier()

    def work(hs_ref):
        @pl.loop(0, 2)
        def _half(h):
            pltpu.sync_copy(src3.at[s].at[pl.ds(h * (NB // 2), NB // 2)],
                            src_v)
            pltpu.sync_copy(dst3.at[s].at[pl.ds(h * (NB // 2), NB // 2)],
                            dst_v)

            pltpu.async_copy(hs_ref.at[src_v.at[0]], rows0, sem0)
            pltpu.async_copy(hs_ref.at[src_v.at[1]], rows1, sem1)

            @pl.loop(0, NB // 4 - 1)
            def _pair(jh):
                j = jh * 2
                pltpu.make_async_copy(hs_ref.at[src_v.at[j]], rows0,
                                      sem0).wait()
                s0 = pltpu.async_copy(rows0, acc.at[dst_v.at[j]], ssem0,
                                      add=True)
                pltpu.make_async_copy(hs_ref.at[src_v.at[j + 1]], rows1,
                                      sem1).wait()
                s1 = pltpu.async_copy(rows1, acc.at[dst_v.at[j + 1]], ssem1,
                                      add=True)
                s0.wait()
                pltpu.async_copy(hs_ref.at[src_v.at[j + 2]], rows0, sem0)
                s1.wait()
                pltpu.async_copy(hs_ref.at[src_v.at[j + 3]], rows1, sem1)

            jl = NB // 2 - 2
            pltpu.make_async_copy(hs_ref.at[src_v.at[jl]], rows0, sem0).wait()
            pltpu.sync_copy(rows0, acc.at[dst_v.at[jl]], add=True)
            pltpu.make_async_copy(hs_ref.at[src_v.at[jl + 1]], rows1,
                                  sem1).wait()
            pltpu.sync_copy(rows1, acc.at[dst_v.at[jl + 1]], add=True)

    @pl.when(c == 0)
    def _w0():
        work(hsA)

    @pl.when(c == 1)
    def _w1():
        work(hsB)

    plsc.subcore_barrier()

    @pl.when(c == 0)
    def _o0():
        pltpu.sync_copy(acc.at[pl.ds(base, ROWS_PER_SUB)],
                        outA.at[pl.ds(base, ROWS_PER_SUB)])

    @pl.when(c == 1)
    def _o1():
        pltpu.sync_copy(acc.at[pl.ds(base, ROWS_PER_SUB)],
                        outB.at[pl.ds(base, ROWS_PER_SUB)])


@functools.cache
def _sc_kernels():
    mesh = plsc.VectorSubcoreMesh(core_axis_name="c", subcore_axis_name="s")
    cp = pltpu.CompilerParams()
    if "needs_layout_passes" in pltpu.CompilerParams.__dataclass_fields__:
        cp = dataclasses.replace(cp, needs_layout_passes=False)
    hist = functools.partial(
        pl.kernel,
        out_type=jax.ShapeDtypeStruct((32, N_PAD), jnp.float32),
        mesh=mesh,
        compiler_params=cp,
        scratch_types=[
            pltpu.VMEM((NB, BB), jnp.int32),
            pltpu.VMEM((N_PAD,), jnp.float32),
        ],
    )(_hist_body)
    conv = functools.partial(
        pl.kernel,
        out_type=[
            jax.ShapeDtypeStruct((N_PAD, 128), jnp.float32),
            jax.ShapeDtypeStruct((N_PAD, 128), jnp.float32),
        ],
        mesh=mesh,
        scratch_types=[
            pltpu.VMEM((NB // 2, BB), jnp.int32),
            pltpu.VMEM((NB // 2, BB), jnp.int32),
            pltpu.VMEM((BB, 128), jnp.float32),
            pltpu.VMEM((BB, 128), jnp.float32),
            pltpu.VMEM_SHARED((N_PAD, 128), jnp.float32),
            pltpu.SemaphoreType.DMA,
            pltpu.SemaphoreType.DMA,
            pltpu.SemaphoreType.DMA,
            pltpu.SemaphoreType.DMA,
        ],
    )(_conv_body)
    return hist, conv


_BM = 1000
_HIGH = jax.lax.Precision.DEFAULT


def _deg_body(p_ref, dis_ref):
    deg = jnp.sum(p_ref[...], axis=0, keepdims=True) + 1.0
    dis_ref[...] = lax.rsqrt(deg)


def _degree_kernel(partials):
    return pl.pallas_call(
        _deg_body,
        out_shape=jax.ShapeDtypeStruct((1, N_PAD), jnp.float32),
    )(partials)


def _m1_body(x_ref, w_ref, dis_ref, a_ref, b_ref):
    hs = jax.lax.dot_general(x_ref[...], w_ref[...], (((1,), (0,)), ((), ())),
                             precision=_HIGH,
                             preferred_element_type=jnp.float32)
    hs = hs * dis_ref[...]
    a_ref[...] = hs[:, :128]
    b_ref[...] = hs[:, 128:]


def _m1_kernel(x, W1, dis_col):
    return pl.pallas_call(
        _m1_body,
        grid=(N_NODES // _BM,),
        in_specs=[
            pl.BlockSpec((_BM, D_IN), lambda i: (i, 0)),
            pl.BlockSpec((D_IN, D_HID), lambda i: (0, 0)),
            pl.BlockSpec((_BM, 1), lambda i: (i, 0)),
        ],
        out_specs=[
            pl.BlockSpec((_BM, 128), lambda i: (i, 0)),
            pl.BlockSpec((_BM, 128), lambda i: (i, 0)),
        ],
        out_shape=[
            jax.ShapeDtypeStruct((N_NODES, 128), jnp.float32),
            jax.ShapeDtypeStruct((N_NODES, 128), jnp.float32),
        ],
    )(x, W1, dis_col)


def _e1_body(aggA_ref, aggB_ref, hsA_ref, hsB_ref, dis_ref, b1_ref, w_ref,
             a_ref, b_ref):
    dis = dis_ref[...]
    hidden = jnp.concatenate(
        [(aggA_ref[...] + hsA_ref[...]) * dis,
         (aggB_ref[...] + hsB_ref[...]) * dis], axis=1)
    hidden = jnp.maximum(hidden + b1_ref[...], 0.0)
    hs2 = jax.lax.dot_general(hidden, w_ref[...], (((1,), (0,)), ((), ())),
                              precision=_HIGH,
                              preferred_element_type=jnp.float32)
    hs2 = hs2 * dis
    a_ref[...] = hs2[:, :128]
    b_ref[...] = hs2[:, 128:]


def _e1_kernel(aggA, aggB, hsA, hsB, dis_col, b1_row, Wcat):
    blk = lambda i: (i, 0)
    return pl.pallas_call(
        _e1_body,
        grid=(N_NODES // _BM,),
        in_specs=[
            pl.BlockSpec((_BM, 128), blk),
            pl.BlockSpec((_BM, 128), blk),
            pl.BlockSpec((_BM, 128), blk),
            pl.BlockSpec((_BM, 128), blk),
            pl.BlockSpec((_BM, 1), blk),
            pl.BlockSpec((1, D_HID), lambda i: (0, 0)),
            pl.BlockSpec((D_HID, 2 * D_LAT), lambda i: (0, 0)),
        ],
        out_specs=[
            pl.BlockSpec((_BM, 128), blk),
            pl.BlockSpec((_BM, 128), blk),
        ],
        out_shape=[
            jax.ShapeDtypeStruct((N_NODES, 128), jnp.float32),
            jax.ShapeDtypeStruct((N_NODES, 128), jnp.float32),
        ],
    )(aggA, aggB, hsA, hsB, dis_col, b1_row, Wcat)


def _e2_body(aggA_ref, aggB_ref, hsA_ref, hsB_ref, dis_ref, bmu_ref, bls_ref,
             eps_ref, mu_ref, ls_ref, z_ref):
    dis = dis_ref[...]
    mu = (aggA_ref[...] + hsA_ref[...]) * dis + bmu_ref[...]
    ls = (aggB_ref[...] + hsB_ref[...]) * dis + bls_ref[...]
    mu_ref[...] = mu
    ls_ref[...] = ls
    z_ref[...] = mu + eps_ref[...] * jnp.exp(ls)


def _e2_kernel(aggA, aggB, hsA, hsB, dis_col, bmu_row, bls_row, eps):
    blk = lambda i: (i, 0)
    return pl.pallas_call(
        _e2_body,
        grid=(N_NODES // _BM,),
        in_specs=[
            pl.BlockSpec((_BM, 128), blk),
            pl.BlockSpec((_BM, 128), blk),
            pl.BlockSpec((_BM, 128), blk),
            pl.BlockSpec((_BM, 128), blk),
            pl.BlockSpec((_BM, 1), blk),
            pl.BlockSpec((1, D_LAT), lambda i: (0, 0)),
            pl.BlockSpec((1, D_LAT), lambda i: (0, 0)),
            pl.BlockSpec((_BM, D_LAT), blk),
        ],
        out_specs=[pl.BlockSpec((_BM, D_LAT), blk)] * 3,
        out_shape=[jax.ShapeDtypeStruct((N_NODES, D_LAT), jnp.float32)] * 3,
    )(aggA, aggB, hsA, hsB, dis_col, bmu_row, bls_row, eps)


def _dec_body(zi_ref, zj_ref, o_ref):
    o_ref[...] = jax.lax.dot_general(
        zi_ref[...], zj_ref[...], (((1,), (1,)), ((), ())),
        precision=jax.lax.Precision.DEFAULT, preferred_element_type=jnp.float32)


_BD = 1024


def _decoder_kernel(z):
    nblk = (N_NODES + _BD - 1) // _BD
    return pl.pallas_call(
        _dec_body,
        grid=(nblk, nblk),
        in_specs=[
            pl.BlockSpec((_BD, D_LAT), lambda i, j: (i, 0)),
            pl.BlockSpec((_BD, D_LAT), lambda i, j: (j, 0)),
        ],
        out_specs=pl.BlockSpec((_BD, _BD), lambda i, j: (i, j)),
        out_shape=jax.ShapeDtypeStruct((N_NODES, N_NODES), jnp.float32),
        compiler_params=pltpu.CompilerParams(
            dimension_semantics=("parallel", "parallel")),
    )(z, z)


def kernel(x, edge_index, W1, b1, W_mu, b_mu, W_ls, b_ls, eps):
    src = edge_index[0].astype(jnp.int32)
    dst = edge_index[1].astype(jnp.int32)

    n_extra = E_PAD - src.shape[0]
    ar = jnp.arange(n_extra, dtype=jnp.int32)
    pad_src = (ar * 131) % N_NODES
    pad_dst = N_NODES + (ar % 16)
    src3 = jnp.concatenate([src, pad_src]).reshape(16, NB, BB)
    dst3 = jnp.concatenate([dst, pad_dst]).reshape(16, NB, BB)

    _hist_kernel, _conv_kernel = _sc_kernels()
    partials = _hist_kernel(dst3)
    dis_row = _degree_kernel(partials)
    dis_col = dis_row.reshape(N_PAD, 1)[:N_NODES]

    hsA1, hsB1 = _m1_kernel(x, W1, dis_col)
    aggA1, aggB1 = _conv_kernel(hsA1, hsB1, src3, dst3)
    Wcat = jnp.concatenate([W_mu, W_ls], axis=1)
    hsA2, hsB2 = _e1_kernel(aggA1, aggB1, hsA1, hsB1,
                            dis_col, b1.reshape(1, -1), Wcat)
    aggA2, aggB2 = _conv_kernel(hsA2, hsB2, src3, dst3)
    mu, logstd, z = _e2_kernel(aggA2, aggB2, hsA2, hsB2,
                               dis_col, b_mu.reshape(1, -1),
                               b_ls.reshape(1, -1), eps)
    adj = _decoder_kernel(z)
    return (adj, mu, logstd)


# speedup vs baseline: 12.6941x; 1.0210x over previous
import dataclasses
import functools

import jax
import jax.numpy as jnp
from jax import lax
from jax.experimental import pallas as pl
from jax.experimental.pallas import tpu as pltpu
from jax.experimental.pallas import tpu_sc as plsc

N_NODES = 10000
D_IN = 256
D_HID = 256
D_LAT = 128

N_PAD = 10112
E_PAD = 163840
NB = 80
BB = 128
ROWS_PER_SUB = N_PAD // 16


def _hist_body(dst3, out, dst_v, hist):
    c = lax.axis_index("c")
    s = lax.axis_index("s")
    wid = c * 16 + s
    pltpu.sync_copy(dst3.at[s], dst_v)

    @pl.loop(0, N_PAD, step=16)
    def _zero(i):
        hist[pl.ds(i, 16)] = jnp.zeros((16,), jnp.float32)

    ones = jnp.ones((16,), jnp.float32)

    @pl.loop(0, NB // 2)
    def _row(j):
        @pl.loop(0, BB, step=16)
        def _vec(k):
            idx = dst_v[j + c * (NB // 2), pl.ds(k, 16)]
            plsc.addupdate_scatter(hist, [idx], ones)

    pltpu.sync_copy(hist, out.at[wid])


def _conv_body(hsA, hsB, src3, dst3, outA, outB, src_v, dst_v, rows0, rows1,
               acc, sem0, sem1, ssem0, ssem1):
    c = lax.axis_index("c")
    s = lax.axis_index("s")
    base = s * ROWS_PER_SUB

    @pl.loop(0, BB)
    def _zr(i):
        @pl.loop(0, 128, step=16)
        def _zc(j):
            rows0[i, pl.ds(j, 16)] = jnp.zeros((16,), jnp.float32)

    @pl.loop(0, ROWS_PER_SUB // BB)
    def _za(k):
        pltpu.sync_copy(rows0, acc.at[pl.ds(base + k * BB, BB)])

    pltpu.sync_copy(
        rows0.at[pl.ds(0, ROWS_PER_SUB % BB)],
        acc.at[pl.ds(base + (ROWS_PER_SUB // BB) * BB, ROWS_PER_SUB % BB)],
    )

    plsc.subcore_barrier()

    def work(hs_ref):
        @pl.loop(0, 2)
        def _half(h):
            pltpu.sync_copy(src3.at[s].at[pl.ds(h * (NB // 2), NB // 2)],
                            src_v)
            pltpu.sync_copy(dst3.at[s].at[pl.ds(h * (NB // 2), NB // 2)],
                            dst_v)

            pltpu.async_copy(hs_ref.at[src_v.at[0]], rows0, sem0)
            pltpu.async_copy(hs_ref.at[src_v.at[1]], rows1, sem1)

            @pl.loop(0, NB // 4 - 1)
            def _pair(jh):
                j = jh * 2
                pltpu.make_async_copy(hs_ref.at[src_v.at[j]], rows0,
                                      sem0).wait()
                s0 = pltpu.async_copy(rows0, acc.at[dst_v.at[j]], ssem0,
                                      add=True)
                pltpu.make_async_copy(hs_ref.at[src_v.at[j + 1]], rows1,
                                      sem1).wait()
                s1 = pltpu.async_copy(rows1, acc.at[dst_v.at[j + 1]], ssem1,
                                      add=True)
                s0.wait()
                pltpu.async_copy(hs_ref.at[src_v.at[j + 2]], rows0, sem0)
                s1.wait()
                pltpu.async_copy(hs_ref.at[src_v.at[j + 3]], rows1, sem1)

            jl = NB // 2 - 2
            pltpu.make_async_copy(hs_ref.at[src_v.at[jl]], rows0, sem0).wait()
            pltpu.sync_copy(rows0, acc.at[dst_v.at[jl]], add=True)
            pltpu.make_async_copy(hs_ref.at[src_v.at[jl + 1]], rows1,
                                  sem1).wait()
            pltpu.sync_copy(rows1, acc.at[dst_v.at[jl + 1]], add=True)

    @pl.when(c == 0)
    def _w0():
        work(hsA)

    @pl.when(c == 1)
    def _w1():
        work(hsB)

    plsc.subcore_barrier()

    @pl.when(c == 0)
    def _o0():
        pltpu.sync_copy(acc.at[pl.ds(base, ROWS_PER_SUB)],
                        outA.at[pl.ds(base, ROWS_PER_SUB)])

    @pl.when(c == 1)
    def _o1():
        pltpu.sync_copy(acc.at[pl.ds(base, ROWS_PER_SUB)],
                        outB.at[pl.ds(base, ROWS_PER_SUB)])


@functools.cache
def _sc_kernels():
    mesh = plsc.VectorSubcoreMesh(core_axis_name="c", subcore_axis_name="s")
    cp = pltpu.CompilerParams()
    if "needs_layout_passes" in pltpu.CompilerParams.__dataclass_fields__:
        cp = dataclasses.replace(cp, needs_layout_passes=False)
    hist = functools.partial(
        pl.kernel,
        out_type=jax.ShapeDtypeStruct((32, N_PAD), jnp.float32),
        mesh=mesh,
        compiler_params=cp,
        scratch_types=[
            pltpu.VMEM((NB, BB), jnp.int32),
            pltpu.VMEM((N_PAD,), jnp.float32),
        ],
    )(_hist_body)
    conv = functools.partial(
        pl.kernel,
        out_type=[
            jax.ShapeDtypeStruct((N_PAD, 128), jnp.float32),
            jax.ShapeDtypeStruct((N_PAD, 128), jnp.float32),
        ],
        mesh=mesh,
        scratch_types=[
            pltpu.VMEM((NB // 2, BB), jnp.int32),
            pltpu.VMEM((NB // 2, BB), jnp.int32),
            pltpu.VMEM((BB, 128), jnp.float32),
            pltpu.VMEM((BB, 128), jnp.float32),
            pltpu.VMEM_SHARED((N_PAD, 128), jnp.float32),
            pltpu.SemaphoreType.DMA,
            pltpu.SemaphoreType.DMA,
            pltpu.SemaphoreType.DMA,
            pltpu.SemaphoreType.DMA,
        ],
    )(_conv_body)
    return hist, conv


_BM = 1000
_HIGH = jax.lax.Precision.DEFAULT


def _deg_body(p_ref, dis_ref):
    deg = jnp.sum(p_ref[...], axis=0, keepdims=True) + 1.0
    dis_ref[...] = lax.rsqrt(deg)


def _degree_kernel(partials):
    return pl.pallas_call(
        _deg_body,
        out_shape=jax.ShapeDtypeStruct((1, N_PAD), jnp.float32),
    )(partials)


def _m1_body(x_ref, w_ref, dis_ref, a_ref, b_ref):
    hs = jax.lax.dot_general(x_ref[...], w_ref[...], (((1,), (0,)), ((), ())),
                             precision=_HIGH,
                             preferred_element_type=jnp.float32)
    hs = hs * dis_ref[...]
    a_ref[...] = hs[:, :128]
    b_ref[...] = hs[:, 128:]


def _m1_kernel(x, W1, dis_col):
    return pl.pallas_call(
        _m1_body,
        grid=(N_NODES // _BM,),
        in_specs=[
            pl.BlockSpec((_BM, D_IN), lambda i: (i, 0)),
            pl.BlockSpec((D_IN, D_HID), lambda i: (0, 0)),
            pl.BlockSpec((_BM, 1), lambda i: (i, 0)),
        ],
        out_specs=[
            pl.BlockSpec((_BM, 128), lambda i: (i, 0)),
            pl.BlockSpec((_BM, 128), lambda i: (i, 0)),
        ],
        out_shape=[
            jax.ShapeDtypeStruct((N_NODES, 128), jnp.float32),
            jax.ShapeDtypeStruct((N_NODES, 128), jnp.float32),
        ],
    )(x, W1, dis_col)


def _e1_body(aggA_ref, aggB_ref, hsA_ref, hsB_ref, dis_ref, b1_ref, w_ref,
             a_ref, b_ref):
    dis = dis_ref[...]
    hidden = jnp.concatenate(
        [(aggA_ref[...] + hsA_ref[...]) * dis,
         (aggB_ref[...] + hsB_ref[...]) * dis], axis=1)
    hidden = jnp.maximum(hidden + b1_ref[...], 0.0)
    hs2 = jax.lax.dot_general(hidden, w_ref[...], (((1,), (0,)), ((), ())),
                              precision=_HIGH,
                              preferred_element_type=jnp.float32)
    hs2 = hs2 * dis
    a_ref[...] = hs2[:, :128]
    b_ref[...] = hs2[:, 128:]


def _e1_kernel(aggA, aggB, hsA, hsB, dis_col, b1_row, Wcat):
    blk = lambda i: (i, 0)
    return pl.pallas_call(
        _e1_body,
        grid=(N_NODES // _BM,),
        in_specs=[
            pl.BlockSpec((_BM, 128), blk),
            pl.BlockSpec((_BM, 128), blk),
            pl.BlockSpec((_BM, 128), blk),
            pl.BlockSpec((_BM, 128), blk),
            pl.BlockSpec((_BM, 1), blk),
            pl.BlockSpec((1, D_HID), lambda i: (0, 0)),
            pl.BlockSpec((D_HID, 2 * D_LAT), lambda i: (0, 0)),
        ],
        out_specs=[
            pl.BlockSpec((_BM, 128), blk),
            pl.BlockSpec((_BM, 128), blk),
        ],
        out_shape=[
            jax.ShapeDtypeStruct((N_NODES, 128), jnp.float32),
            jax.ShapeDtypeStruct((N_NODES, 128), jnp.float32),
        ],
    )(aggA, aggB, hsA, hsB, dis_col, b1_row, Wcat)


def _e2_body(aggA_ref, aggB_ref, hsA_ref, hsB_ref, dis_ref, bmu_ref, bls_ref,
             eps_ref, mu_ref, ls_ref, z_ref):
    dis = dis_ref[...]
    mu = (aggA_ref[...] + hsA_ref[...]) * dis + bmu_ref[...]
    ls = (aggB_ref[...] + hsB_ref[...]) * dis + bls_ref[...]
    mu_ref[...] = mu
    ls_ref[...] = ls
    z_ref[...] = (mu + eps_ref[...] * jnp.exp(ls)).astype(jnp.bfloat16)


def _e2_kernel(aggA, aggB, hsA, hsB, dis_col, bmu_row, bls_row, eps):
    blk = lambda i: (i, 0)
    return pl.pallas_call(
        _e2_body,
        grid=(N_NODES // _BM,),
        in_specs=[
            pl.BlockSpec((_BM, 128), blk),
            pl.BlockSpec((_BM, 128), blk),
            pl.BlockSpec((_BM, 128), blk),
            pl.BlockSpec((_BM, 128), blk),
            pl.BlockSpec((_BM, 1), blk),
            pl.BlockSpec((1, D_LAT), lambda i: (0, 0)),
            pl.BlockSpec((1, D_LAT), lambda i: (0, 0)),
            pl.BlockSpec((_BM, D_LAT), blk),
        ],
        out_specs=[pl.BlockSpec((_BM, D_LAT), blk)] * 3,
        out_shape=[jax.ShapeDtypeStruct((N_NODES, D_LAT), jnp.float32)] * 2
        + [jax.ShapeDtypeStruct((N_NODES, D_LAT), jnp.bfloat16)],
    )(aggA, aggB, hsA, hsB, dis_col, bmu_row, bls_row, eps)


def _dec_body(zi_ref, zj_ref, o_ref):
    o_ref[...] = jax.lax.dot_general(
        zi_ref[...], zj_ref[...], (((1,), (1,)), ((), ())),
        precision=jax.lax.Precision.DEFAULT, preferred_element_type=jnp.float32)


_BD = 1024


def _decoder_kernel(z):
    nblk = (N_NODES + _BD - 1) // _BD
    return pl.pallas_call(
        _dec_body,
        grid=(nblk, nblk),
        in_specs=[
            pl.BlockSpec((_BD, D_LAT), lambda i, j: (i, 0)),
            pl.BlockSpec((_BD, D_LAT), lambda i, j: (j, 0)),
        ],
        out_specs=pl.BlockSpec((_BD, _BD), lambda i, j: (i, j)),
        out_shape=jax.ShapeDtypeStruct((N_NODES, N_NODES), jnp.float32),
        compiler_params=pltpu.CompilerParams(
            dimension_semantics=("parallel", "parallel")),
    )(z, z)


def kernel(x, edge_index, W1, b1, W_mu, b_mu, W_ls, b_ls, eps):
    src = edge_index[0].astype(jnp.int32)
    dst = edge_index[1].astype(jnp.int32)

    n_extra = E_PAD - src.shape[0]
    ar = jnp.arange(n_extra, dtype=jnp.int32)
    pad_src = (ar * 131) % N_NODES
    pad_dst = N_NODES + (ar % 16)
    src3 = jnp.concatenate([src, pad_src]).reshape(16, NB, BB)
    dst3 = jnp.concatenate([dst, pad_dst]).reshape(16, NB, BB)

    _hist_kernel, _conv_kernel = _sc_kernels()
    partials = _hist_kernel(dst3)
    dis_row = _degree_kernel(partials)
    dis_col = dis_row.reshape(N_PAD, 1)[:N_NODES]

    hsA1, hsB1 = _m1_kernel(x, W1, dis_col)
    aggA1, aggB1 = _conv_kernel(hsA1, hsB1, src3, dst3)
    Wcat = jnp.concatenate([W_mu, W_ls], axis=1)
    hsA2, hsB2 = _e1_kernel(aggA1, aggB1, hsA1, hsB1,
                            dis_col, b1.reshape(1, -1), Wcat)
    aggA2, aggB2 = _conv_kernel(hsA2, hsB2, src3, dst3)
    mu, logstd, z = _e2_kernel(aggA2, aggB2, hsA2, hsB2,
                               dis_col, b_mu.reshape(1, -1),
                               b_ls.reshape(1, -1), eps)
    adj = _decoder_kernel(z)
    return (adj, mu, logstd)
